# Initial kernel scaffold; baseline (speedup 1.0000x reference)
import jax
import jax.numpy as jnp
from jax.experimental import pallas as pl


def kernel(tgt_values, tgt_positions, edge_src, edge_dst, coord_tab, pos_tab, val_tab, ln1_s, ln1_b, Wq, Wk, Wv, Wo, ln2_s, ln2_b, W1, W2, gen_ln_s, gen_ln_b, Wgen):
    raise NotImplementedError("write your pallas kernel here")


# speedup vs baseline: 15.3002x; 15.3002x over previous
import functools

import jax
import jax.numpy as jnp
from jax import lax
from jax.experimental import pallas as pl
from jax.experimental.pallas import tpu as pltpu
from jax.experimental.pallas import tpu_sc as plsc

N = 10000
E = 320000
D = 128
H = 8
DK = 16
V = 512
P = 64
FF = 512

R = 400
GRID = N // R

NC = 2
NS = 16
NW = NC * NS
EW = E // NW
B = 40
NB = EW // B
N_PAD = 10240
NZ = N_PAD // 16
RC = 64
SCALE = 1.0 / (DK ** 0.5)


def _ln(x, s, b):
    m = jnp.mean(x, axis=-1, keepdims=True)
    var = jnp.mean(x * x, axis=-1, keepdims=True) - m * m
    return (x - m) * jax.lax.rsqrt(var + 1e-5) * s + b


def _qkv(xn, wq_ref, wk_ref, wv_ref, q_out, kv_out):
    q_out[...] = jnp.dot(xn, wq_ref[...], preferred_element_type=jnp.float32)
    kv_out[:, 0:D] = jnp.dot(xn, wk_ref[...], preferred_element_type=jnp.float32)
    kv_out[:, D:2 * D] = jnp.dot(xn, wv_ref[...], preferred_element_type=jnp.float32)


def _post(x, wv_ref, z_ref, wo_ref, ln2s_ref, ln2b_ref, w1_ref, w2_ref):
    wv = wv_ref[0] + wv_ref[1]
    z = z_ref[0] + z_ref[1]
    r = 1.0 / (z + 1e-9)
    col = lax.broadcasted_iota(jnp.int32, (H, D), 1) // DK
    row = lax.broadcasted_iota(jnp.int32, (H, D), 0)
    expand = (col == row).astype(jnp.float32)
    rrep = jnp.dot(r, expand, preferred_element_type=jnp.float32)
    o = jnp.dot(wv * rrep, wo_ref[...], preferred_element_type=jnp.float32)
    x1 = x + o
    xn2 = _ln(x1, ln2s_ref[...], ln2b_ref[...])
    f = jnp.dot(
        jax.nn.relu(jnp.dot(xn2, w1_ref[...], preferred_element_type=jnp.float32)),
        w2_ref[...], preferred_element_type=jnp.float32)
    return x1 + f


def _embed_qkv_body(vals_ref, pos_ref, coord_ref, ptab_ref, vtab_ref,
                    ln1s_ref, ln1b_ref, wq_ref, wk_ref, wv_ref,
                    x_out, q_out, kv_out):
    vals = vals_ref[...]
    pos = pos_ref[...]
    c = pos % 3
    p = pos // 3
    x = jnp.where(c == 0, coord_ref[0:1, :],
                  jnp.where(c == 1, coord_ref[1:2, :], coord_ref[2:3, :]))
    oh_p = (p == lax.broadcasted_iota(jnp.int32, (R, P), 1)).astype(jnp.float32)
    x = x + jnp.dot(oh_p, ptab_ref[...], preferred_element_type=jnp.float32)
    oh_v = (vals == lax.broadcasted_iota(jnp.int32, (R, V), 1)).astype(jnp.float32)
    x = x + jnp.dot(oh_v, vtab_ref[...], preferred_element_type=jnp.float32)
    x_out[...] = x
    xn = _ln(x, ln1s_ref[...], ln1b_ref[...])
    _qkv(xn, wq_ref, wk_ref, wv_ref, q_out, kv_out)


def _post_qkv_body(x_ref, wv_ref, z_ref, wo_ref, ln2s_ref, ln2b_ref,
                   w1_ref, w2_ref, ln1s_ref, ln1b_ref, wq_ref, wk_ref, wv2_ref,
                   x_out, q_out, kv_out):
    x2 = _post(x_ref[...], wv_ref, z_ref, wo_ref, ln2s_ref, ln2b_ref,
               w1_ref, w2_ref)
    x_out[...] = x2
    xn = _ln(x2, ln1s_ref[...], ln1b_ref[...])
    _qkv(xn, wq_ref, wk_ref, wv2_ref, q_out, kv_out)


def _post_gen_body(x_ref, wv_ref, z_ref, wo_ref, ln2s_ref, ln2b_ref,
                   w1_ref, w2_ref, glns_ref, glnb_ref, wgen_ref, out_ref):
    x2 = _post(x_ref[...], wv_ref, z_ref, wo_ref, ln2s_ref, ln2b_ref,
               w1_ref, w2_ref)
    xg = _ln(x2, glns_ref[...], glnb_ref[...])
    logits = jnp.dot(xg, wgen_ref[...], preferred_element_type=jnp.float32)
    m = jnp.max(logits, axis=-1, keepdims=True)
    lse = m + jnp.log(jnp.sum(jnp.exp(logits - m), axis=-1, keepdims=True))
    out_ref[...] = logits - lse


def _full(shape):
    return pl.BlockSpec(shape, lambda i: tuple(0 for _ in shape))


_ROW = pl.BlockSpec((R, D), lambda i: (i, 0))
_ROW_KV = pl.BlockSpec((R, 2 * D), lambda i: (i, 0))
_ROW_WV = pl.BlockSpec((NC, R, D), lambda i: (0, i, 0))
_ROW_Z = pl.BlockSpec((NC, R, H), lambda i: (0, i, 0))
_ROW_IDX = pl.BlockSpec((R, 1), lambda i: (i, 0))

_embed_qkv = pl.pallas_call(
    _embed_qkv_body,
    grid=(GRID,),
    in_specs=[_ROW_IDX, _ROW_IDX, _full((8, D)), _full((P, D)), _full((V, D)),
              _full((1, D)), _full((1, D)),
              _full((D, D)), _full((D, D)), _full((D, D))],
    out_specs=[_ROW, _ROW, _ROW_KV],
    out_shape=[jax.ShapeDtypeStruct((N, D), jnp.float32),
               jax.ShapeDtypeStruct((N, D), jnp.float32),
               jax.ShapeDtypeStruct((N, 2 * D), jnp.float32)],
)

_post_qkv = pl.pallas_call(
    _post_qkv_body,
    grid=(GRID,),
    in_specs=[_ROW, _ROW_WV, _ROW_Z, _full((D, D)), _full((1, D)), _full((1, D)),
              _full((D, FF)), _full((FF, D)),
              _full((1, D)), _full((1, D)),
              _full((D, D)), _full((D, D)), _full((D, D))],
    out_specs=[_ROW, _ROW, _ROW_KV],
    out_shape=[jax.ShapeDtypeStruct((N, D), jnp.float32),
               jax.ShapeDtypeStruct((N, D), jnp.float32),
               jax.ShapeDtypeStruct((N, 2 * D), jnp.float32)],
)

_post_gen = pl.pallas_call(
    _post_gen_body,
    grid=(GRID,),
    in_specs=[_ROW, _ROW_WV, _ROW_Z, _full((D, D)), _full((1, D)), _full((1, D)),
              _full((D, FF)), _full((FF, D)),
              _full((1, D)), _full((1, D)), _full((D, V))],
    out_specs=pl.BlockSpec((R, V), lambda i: (i, 0)),
    out_shape=jax.ShapeDtypeStruct((N, V), jnp.float32),
)


def _edge_body(q_hbm, kv_hbm, src_hbm, dst_hbm, wv_hbm, z_hbm,
               src_idx, dst_idx, dst_pad, zidx, kvrows, qrows,
               wvrows, zrows, chunk, pb, ztmp,
               acc_wv, acc_z, sem1, sem2):
    cid = lax.axis_index("c")
    sid = lax.axis_index("s")
    wid = cid * NS + sid
    zeros16 = jnp.zeros((16,), jnp.float32)
    lanes = lax.broadcasted_iota(jnp.int32, (16,), 0)

    def _zero_row(rr, _):
        for cc in range(D // 16):
            chunk[rr, pl.ds(cc * 16, 16)] = zeros16
        return 0
    lax.fori_loop(0, RC, _zero_row, 0)
    for t in range(N_PAD // NS // RC):
        pltpu.sync_copy(chunk, acc_wv.at[pl.ds(sid * (N_PAD // NS) + t * RC, RC)])
    pltpu.sync_copy(chunk.at[pl.ds(0, NZ // NS)],
                    acc_z.at[pl.ds(sid * (NZ // NS), NZ // NS)])
    ztmp[pl.ds(0, 16)] = zeros16
    ztmp[pl.ds(16, 16)] = zeros16
    plsc.subcore_barrier()

    def _batch(j, _):
        base = wid * EW + j * B
        pltpu.sync_copy(src_hbm.at[pl.ds(base, B)], src_idx)
        pltpu.sync_copy(dst_hbm.at[pl.ds(base, B)], dst_idx)
        cp1 = pltpu.async_copy(kv_hbm.at[src_idx], kvrows, sem1)
        cp2 = pltpu.async_copy(q_hbm.at[dst_idx], qrows, sem2)
        for c in range(-(-B // 16)):
            cc = min(c * 16, B - 16)
            dv = dst_idx[pl.ds(cc, 16)]
            zidx[pl.ds(cc, 16)] = lax.shift_right_logical(dv, 4)
            dst_pad[pl.ds(cc, 16)] = dv
        dst_pad[pl.ds(B, 16)] = jnp.zeros((16,), jnp.int32)

        def _zrow_zero(e, _):
            for cc in range(D // 16):
                zrows[e, pl.ds(cc * 16, 16)] = zeros16
            return 0
        lax.fori_loop(0, B, _zrow_zero, 0)
        cp1.wait()
        cp2.wait()

        def _edge(e, _):
            d = dst_pad[pl.ds(e, 16)][0]
            svec = zeros16
            for h in range(H):
                kvec = kvrows[e, pl.ds(h * DK, 16)]
                qvec = qrows[e, pl.ds(h * DK, 16)]
                r = kvec * qvec
                for step, off in ((0, 8), (16, 4), (32, 2), (48, 1)):
                    pb[pl.ds(step, 16)] = r
                    r = r + pb[pl.ds(step + off, 16)]
                svec = jnp.where(lanes == h, r[0], svec)
            scv = jnp.exp(jnp.clip(svec * SCALE, -10.0, 10.0))
            zrow = jnp.where(lanes < H, scv, 0.0)
            ztmp[pl.ds(8, 16)] = zrow
            zrow_hi = ztmp[pl.ds(0, 16)]
            o = jnp.bitwise_and(d, 15) * H
            hi = o > D - 16
            off = jnp.where(hi, D - 16, o)
            hf = jnp.where(hi, 1.0, 0.0)
            zrows[e, pl.ds(off, 16)] = zrow + (zrow_hi - zrow) * hf
            for h in range(H):
                sch = scv[h]
                vvec = kvrows[e, pl.ds(D + h * DK, 16)]
                wvrows[e, pl.ds(h * DK, 16)] = vvec * jnp.broadcast_to(sch, (16,))
            return 0
        lax.fori_loop(0, B, _edge, 0)
        pltpu.sync_copy(wvrows, acc_wv.at[dst_idx], add=True)
        pltpu.sync_copy(zrows, acc_z.at[zidx], add=True)
        return 0

    lax.fori_loop(0, NB, _batch, 0)
    plsc.subcore_barrier()

    for t in range(N_PAD // NS // RC):
        r0 = sid * (N_PAD // NS) + t * RC
        pltpu.sync_copy(acc_wv.at[pl.ds(r0, RC)], chunk)
        pltpu.sync_copy(chunk, wv_hbm.at[cid, pl.ds(r0, RC)])
    z0 = sid * (NZ // NS)
    pltpu.sync_copy(acc_z.at[pl.ds(z0, NZ // NS)], chunk.at[pl.ds(0, NZ // NS)])
    pltpu.sync_copy(chunk.at[pl.ds(0, NZ // NS)], z_hbm.at[cid, pl.ds(z0, NZ // NS)])


@functools.lru_cache(maxsize=1)
def _edge_attention():
    return pl.kernel(
        _edge_body,
        out_type=(jax.ShapeDtypeStruct((NC, N_PAD, D), jnp.float32),
                  jax.ShapeDtypeStruct((NC, NZ, D), jnp.float32)),
        mesh=plsc.VectorSubcoreMesh(core_axis_name="c", subcore_axis_name="s",
                                    num_cores=NC, num_subcores=NS),
        scratch_types=[
            pltpu.VMEM((B,), jnp.int32),
            pltpu.VMEM((B,), jnp.int32),
            pltpu.VMEM((B + 16,), jnp.int32),
            pltpu.VMEM((B,), jnp.int32),
            pltpu.VMEM((B, 2 * D), jnp.float32),
            pltpu.VMEM((B, D), jnp.float32),
            pltpu.VMEM((B, D), jnp.float32),
            pltpu.VMEM((B, D), jnp.float32),
            pltpu.VMEM((RC, D), jnp.float32),
            pltpu.VMEM((80,), jnp.float32),
            pltpu.VMEM((32,), jnp.float32),
            pltpu.VMEM_SHARED((N_PAD, D), jnp.float32),
            pltpu.VMEM_SHARED((NZ, D), jnp.float32),
            pltpu.SemaphoreType.DMA,
            pltpu.SemaphoreType.DMA,
        ],
    )


def kernel(tgt_values, tgt_positions, edge_src, edge_dst, coord_tab, pos_tab,
           val_tab, ln1_s, ln1_b, Wq, Wk, Wv, Wo, ln2_s, ln2_b, W1, W2,
           gen_ln_s, gen_ln_b, Wgen):
    vals2 = tgt_values.astype(jnp.int32).reshape(N, 1)
    pos2 = tgt_positions.astype(jnp.int32).reshape(N, 1)
    src = edge_src.astype(jnp.int32)
    dst = edge_dst.astype(jnp.int32)
    coordp = jnp.zeros((8, D), jnp.float32).at[0:3].set(coord_tab)

    x0, q0, kv0 = _embed_qkv(
        vals2, pos2, coordp, pos_tab, val_tab,
        ln1_s[0].reshape(1, D), ln1_b[0].reshape(1, D), Wq[0], Wk[0], Wv[0])
    edge_attention = _edge_attention()
    wv0, z0 = edge_attention(q0, kv0, src, dst)
    zp0 = z0.reshape(NC, N_PAD, H)
    x1, q1, kv1 = _post_qkv(
        x0, wv0, zp0, Wo[0], ln2_s[0].reshape(1, D), ln2_b[0].reshape(1, D),
        W1[0], W2[0],
        ln1_s[1].reshape(1, D), ln1_b[1].reshape(1, D), Wq[1], Wk[1], Wv[1])
    wv1, z1 = edge_attention(q1, kv1, src, dst)
    zp1 = z1.reshape(NC, N_PAD, H)
    out = _post_gen(
        x1, wv1, zp1, Wo[1], ln2_s[1].reshape(1, D), ln2_b[1].reshape(1, D),
        W1[1], W2[1],
        gen_ln_s.reshape(1, D), gen_ln_b.reshape(1, D), Wgen)
    return out


# speedup vs baseline: 19.0446x; 1.2447x over previous
import functools

import jax
import jax.numpy as jnp
from jax import lax
from jax.experimental import pallas as pl
from jax.experimental.pallas import tpu as pltpu
from jax.experimental.pallas import tpu_sc as plsc

N = 10000
E = 320000
D = 128
H = 8
DK = 16
V = 512
P = 64
FF = 512

R = 400
GRID = N // R

NC = 2
NS = 16
NW = NC * NS
EW = E // NW
B = 40
NB = EW // B
N_PAD = 10240
NZ = N_PAD // 16
RC = 8
SCALE = 1.0 / (DK ** 0.5)


def _ln(x, s, b):
    m = jnp.mean(x, axis=-1, keepdims=True)
    var = jnp.mean(x * x, axis=-1, keepdims=True) - m * m
    return (x - m) * jax.lax.rsqrt(var + 1e-5) * s + b


def _qkv(xn, wq_ref, wk_ref, wv_ref, q_out, kv_out):
    q_out[...] = jnp.dot(xn, wq_ref[...], preferred_element_type=jnp.float32)
    kv_out[:, 0:D] = jnp.dot(xn, wk_ref[...], preferred_element_type=jnp.float32)
    kv_out[:, D:2 * D] = jnp.dot(xn, wv_ref[...], preferred_element_type=jnp.float32)


def _post(x, wv_ref, z_ref, wo_ref, ln2s_ref, ln2b_ref, w1_ref, w2_ref):
    wv = wv_ref[0] + wv_ref[1]
    z = z_ref[0] + z_ref[1]
    r = 1.0 / (z + 1e-9)
    col = lax.broadcasted_iota(jnp.int32, (H, D), 1) // DK
    row = lax.broadcasted_iota(jnp.int32, (H, D), 0)
    expand = (col == row).astype(jnp.float32)
    rrep = jnp.dot(r, expand, preferred_element_type=jnp.float32)
    o = jnp.dot(wv * rrep, wo_ref[...], preferred_element_type=jnp.float32)
    x1 = x + o
    xn2 = _ln(x1, ln2s_ref[...], ln2b_ref[...])
    f = jnp.dot(
        jax.nn.relu(jnp.dot(xn2, w1_ref[...], preferred_element_type=jnp.float32)),
        w2_ref[...], preferred_element_type=jnp.float32)
    return x1 + f


def _embed_qkv_body(vals_ref, pos_ref, coord_ref, ptab_ref, vtab_ref,
                    ln1s_ref, ln1b_ref, wq_ref, wk_ref, wv_ref,
                    x_out, q_out, kv_out):
    vals = vals_ref[...]
    pos = pos_ref[...]
    c = pos % 3
    p = pos // 3
    x = jnp.where(c == 0, coord_ref[0:1, :],
                  jnp.where(c == 1, coord_ref[1:2, :], coord_ref[2:3, :]))
    oh_p = (p == lax.broadcasted_iota(jnp.int32, (R, P), 1)).astype(jnp.float32)
    x = x + jnp.dot(oh_p, ptab_ref[...], preferred_element_type=jnp.float32)
    oh_v = (vals == lax.broadcasted_iota(jnp.int32, (R, V), 1)).astype(jnp.float32)
    x = x + jnp.dot(oh_v, vtab_ref[...], preferred_element_type=jnp.float32)
    x_out[...] = x
    xn = _ln(x, ln1s_ref[...], ln1b_ref[...])
    _qkv(xn, wq_ref, wk_ref, wv_ref, q_out, kv_out)


def _post_qkv_body(x_ref, wv_ref, z_ref, wo_ref, ln2s_ref, ln2b_ref,
                   w1_ref, w2_ref, ln1s_ref, ln1b_ref, wq_ref, wk_ref, wv2_ref,
                   x_out, q_out, kv_out):
    x2 = _post(x_ref[...], wv_ref, z_ref, wo_ref, ln2s_ref, ln2b_ref,
               w1_ref, w2_ref)
    x_out[...] = x2
    xn = _ln(x2, ln1s_ref[...], ln1b_ref[...])
    _qkv(xn, wq_ref, wk_ref, wv2_ref, q_out, kv_out)


def _post_gen_body(x_ref, wv_ref, z_ref, wo_ref, ln2s_ref, ln2b_ref,
                   w1_ref, w2_ref, glns_ref, glnb_ref, wgen_ref, out_ref):
    x2 = _post(x_ref[...], wv_ref, z_ref, wo_ref, ln2s_ref, ln2b_ref,
               w1_ref, w2_ref)
    xg = _ln(x2, glns_ref[...], glnb_ref[...])
    logits = jnp.dot(xg, wgen_ref[...], preferred_element_type=jnp.float32)
    m = jnp.max(logits, axis=-1, keepdims=True)
    lse = m + jnp.log(jnp.sum(jnp.exp(logits - m), axis=-1, keepdims=True))
    out_ref[...] = logits - lse


def _full(shape):
    return pl.BlockSpec(shape, lambda i: tuple(0 for _ in shape))


_ROW = pl.BlockSpec((R, D), lambda i: (i, 0))
_ROW_KV = pl.BlockSpec((R, 2 * D), lambda i: (i, 0))
_ROW_WV = pl.BlockSpec((NC, R, D), lambda i: (0, i, 0))
_ROW_Z = pl.BlockSpec((NC, R, H), lambda i: (0, i, 0))
_ROW_IDX = pl.BlockSpec((R, 1), lambda i: (i, 0))

_embed_qkv = pl.pallas_call(
    _embed_qkv_body,
    grid=(GRID,),
    in_specs=[_ROW_IDX, _ROW_IDX, _full((8, D)), _full((P, D)), _full((V, D)),
              _full((1, D)), _full((1, D)),
              _full((D, D)), _full((D, D)), _full((D, D))],
    out_specs=[_ROW, _ROW, _ROW_KV],
    out_shape=[jax.ShapeDtypeStruct((N, D), jnp.float32),
               jax.ShapeDtypeStruct((N, D), jnp.float32),
               jax.ShapeDtypeStruct((N, 2 * D), jnp.float32)],
)

_post_qkv = pl.pallas_call(
    _post_qkv_body,
    grid=(GRID,),
    in_specs=[_ROW, _ROW_WV, _ROW_Z, _full((D, D)), _full((1, D)), _full((1, D)),
              _full((D, FF)), _full((FF, D)),
              _full((1, D)), _full((1, D)),
              _full((D, D)), _full((D, D)), _full((D, D))],
    out_specs=[_ROW, _ROW, _ROW_KV],
    out_shape=[jax.ShapeDtypeStruct((N, D), jnp.float32),
               jax.ShapeDtypeStruct((N, D), jnp.float32),
               jax.ShapeDtypeStruct((N, 2 * D), jnp.float32)],
)

_post_gen = pl.pallas_call(
    _post_gen_body,
    grid=(GRID,),
    in_specs=[_ROW, _ROW_WV, _ROW_Z, _full((D, D)), _full((1, D)), _full((1, D)),
              _full((D, FF)), _full((FF, D)),
              _full((1, D)), _full((1, D)), _full((D, V))],
    out_specs=pl.BlockSpec((R, V), lambda i: (i, 0)),
    out_shape=jax.ShapeDtypeStruct((N, V), jnp.float32),
)


def _edge_body(q_hbm, kv_hbm, src_hbm, dst_hbm, wv_hbm, z_hbm,
               src_idx0, dst_idx0, dst_pad0, zidx0, kvrows0, qrows0,
               src_idx1, dst_idx1, dst_pad1, zidx1, kvrows1, qrows1,
               wvrows, zrows, chunk, pb, ztmp,
               acc_wv, acc_z,
               ksem0, qsem0, ksem1, qsem1, isem0, isem1):
    cid = lax.axis_index("c")
    sid = lax.axis_index("s")
    wid = cid * NS + sid
    zeros16 = jnp.zeros((16,), jnp.float32)
    lanes = lax.broadcasted_iota(jnp.int32, (16,), 0)

    bufs = ((src_idx0, dst_idx0, dst_pad0, zidx0, kvrows0, qrows0,
             ksem0, qsem0, isem0),
            (src_idx1, dst_idx1, dst_pad1, zidx1, kvrows1, qrows1,
             ksem1, qsem1, isem1))

    def _zero_row(rr, _):
        for cc in range(D // 16):
            chunk[rr, pl.ds(cc * 16, 16)] = zeros16
        return 0
    lax.fori_loop(0, RC, _zero_row, 0)

    def _zero_wv(t, _):
        pltpu.sync_copy(chunk, acc_wv.at[pl.ds(sid * (N_PAD // NS) + t * RC, RC)])
        return 0
    lax.fori_loop(0, N_PAD // NS // RC, _zero_wv, 0)

    def _zero_z(t, _):
        pltpu.sync_copy(chunk, acc_z.at[pl.ds(sid * (NZ // NS) + t * RC, RC)])
        return 0
    lax.fori_loop(0, NZ // NS // RC, _zero_z, 0)
    ztmp[pl.ds(0, 16)] = zeros16
    ztmp[pl.ds(16, 16)] = zeros16
    plsc.subcore_barrier()

    def _issue_idx(b, j):
        base = wid * EW + j * B
        pltpu.async_copy(src_hbm.at[pl.ds(base, B)], bufs[b][0], bufs[b][8])
        pltpu.async_copy(dst_hbm.at[pl.ds(base, B)], bufs[b][1], bufs[b][8])

    def _wait_idx(b):
        pltpu.make_async_copy(src_hbm.at[pl.ds(0, B)], bufs[b][0], bufs[b][8]).wait()
        pltpu.make_async_copy(dst_hbm.at[pl.ds(0, B)], bufs[b][1], bufs[b][8]).wait()

    def _issue_gather(b):
        pltpu.async_copy(kv_hbm.at[bufs[b][0]], bufs[b][4], bufs[b][6])
        pltpu.async_copy(q_hbm.at[bufs[b][1]], bufs[b][5], bufs[b][7])

    def _wait_gather(b):
        pltpu.make_async_copy(kv_hbm.at[bufs[b][0]], bufs[b][4], bufs[b][6]).wait()
        pltpu.make_async_copy(q_hbm.at[bufs[b][1]], bufs[b][5], bufs[b][7]).wait()

    def _compute_scatter(b, j):
        src_b, dst_b, pad_b, zix_b, kvrows_b, qrows_b = bufs[b][:6]
        for c in range(-(-B // 16)):
            cc = min(c * 16, B - 16)
            dv = dst_b[pl.ds(cc, 16)]
            zix_b[pl.ds(cc, 16)] = lax.shift_right_logical(dv, 4)
            pad_b[pl.ds(cc, 16)] = dv
        pad_b[pl.ds(B, 16)] = jnp.zeros((16,), jnp.int32)

        def _zrow_zero(e, _):
            for cc in range(D // 16):
                zrows[e, pl.ds(cc * 16, 16)] = zeros16
            return 0
        lax.fori_loop(0, B, _zrow_zero, 0)

        def _edge(e, _):
            d = pad_b[pl.ds(e, 16)][0]
            svec = zeros16
            for h in range(H):
                kvec = kvrows_b[e, pl.ds(h * DK, 16)]
                qvec = qrows_b[e, pl.ds(h * DK, 16)]
                r = kvec * qvec
                for step, off in ((0, 8), (16, 4), (32, 2), (48, 1)):
                    pb[pl.ds(step, 16)] = r
                    r = r + pb[pl.ds(step + off, 16)]
                svec = jnp.where(lanes == h, r[0], svec)
            scv = jnp.exp(jnp.clip(svec * SCALE, -10.0, 10.0))
            zrow = jnp.where(lanes < H, scv, 0.0)
            ztmp[pl.ds(8, 16)] = zrow
            zrow_hi = ztmp[pl.ds(0, 16)]
            o = jnp.bitwise_and(d, 15) * H
            hi = o > D - 16
            off = jnp.where(hi, D - 16, o)
            hf = jnp.where(hi, 1.0, 0.0)
            zrows[e, pl.ds(off, 16)] = zrow + (zrow_hi - zrow) * hf
            for h in range(H):
                sch = scv[h]
                vvec = kvrows_b[e, pl.ds(D + h * DK, 16)]
                wvrows[e, pl.ds(h * DK, 16)] = vvec * jnp.broadcast_to(sch, (16,))
            return 0
        lax.fori_loop(0, B, _edge, 0)
        pltpu.sync_copy(wvrows, acc_wv.at[dst_b], add=True)
        pltpu.sync_copy(zrows, acc_z.at[zix_b], add=True)

    _issue_idx(0, 0)
    _wait_idx(0)
    _issue_gather(0)
    _issue_idx(1, 1)

    def _super(g, _):
        for b in range(2):
            j = 2 * g + b
            @pl.when(j + 1 < NB)
            def _():
                _wait_idx(1 - b)
                _issue_gather(1 - b)
            _wait_gather(b)
            _compute_scatter(b, j)
            @pl.when(j + 2 < NB)
            def _():
                _issue_idx(b, j + 2)
        return 0

    lax.fori_loop(0, NB // 2, _super, 0)
    plsc.subcore_barrier()

    def _dump_wv(t, _):
        r0 = sid * (N_PAD // NS) + t * RC
        pltpu.sync_copy(acc_wv.at[pl.ds(r0, RC)], chunk)
        pltpu.sync_copy(chunk, wv_hbm.at[cid, pl.ds(r0, RC)])
        return 0
    lax.fori_loop(0, N_PAD // NS // RC, _dump_wv, 0)

    def _dump_z(t, _):
        r0 = sid * (NZ // NS) + t * RC
        pltpu.sync_copy(acc_z.at[pl.ds(r0, RC)], chunk)
        pltpu.sync_copy(chunk, z_hbm.at[cid, pl.ds(r0, RC)])
        return 0
    lax.fori_loop(0, NZ // NS // RC, _dump_z, 0)


@functools.lru_cache(maxsize=1)
def _edge_attention():
    return pl.kernel(
        _edge_body,
        out_type=(jax.ShapeDtypeStruct((NC, N_PAD, D), jnp.float32),
                  jax.ShapeDtypeStruct((NC, NZ, D), jnp.float32)),
        mesh=plsc.VectorSubcoreMesh(core_axis_name="c", subcore_axis_name="s",
                                    num_cores=NC, num_subcores=NS),
        scratch_types=(
            [pltpu.VMEM((B,), jnp.int32),
             pltpu.VMEM((B,), jnp.int32),
             pltpu.VMEM((B + 16,), jnp.int32),
             pltpu.VMEM((B,), jnp.int32),
             pltpu.VMEM((B, 2 * D), jnp.float32),
             pltpu.VMEM((B, D), jnp.float32)] * 2
            + [
                pltpu.VMEM((B, D), jnp.float32),
                pltpu.VMEM((B, D), jnp.float32),
                pltpu.VMEM((RC, D), jnp.float32),
                pltpu.VMEM((80,), jnp.float32),
                pltpu.VMEM((32,), jnp.float32),
                pltpu.VMEM_SHARED((N_PAD, D), jnp.float32),
                pltpu.VMEM_SHARED((NZ, D), jnp.float32),
                pltpu.SemaphoreType.DMA,
                pltpu.SemaphoreType.DMA,
                pltpu.SemaphoreType.DMA,
                pltpu.SemaphoreType.DMA,
                pltpu.SemaphoreType.DMA,
                pltpu.SemaphoreType.DMA,
            ]),
    )


def kernel(tgt_values, tgt_positions, edge_src, edge_dst, coord_tab, pos_tab,
           val_tab, ln1_s, ln1_b, Wq, Wk, Wv, Wo, ln2_s, ln2_b, W1, W2,
           gen_ln_s, gen_ln_b, Wgen):
    vals2 = tgt_values.astype(jnp.int32).reshape(N, 1)
    pos2 = tgt_positions.astype(jnp.int32).reshape(N, 1)
    src = edge_src.astype(jnp.int32)
    dst = edge_dst.astype(jnp.int32)
    coordp = jnp.zeros((8, D), jnp.float32).at[0:3].set(coord_tab)

    x0, q0, kv0 = _embed_qkv(
        vals2, pos2, coordp, pos_tab, val_tab,
        ln1_s[0].reshape(1, D), ln1_b[0].reshape(1, D), Wq[0], Wk[0], Wv[0])
    edge_attention = _edge_attention()
    wv0, z0 = edge_attention(q0, kv0, src, dst)
    zp0 = z0.reshape(NC, N_PAD, H)
    x1, q1, kv1 = _post_qkv(
        x0, wv0, zp0, Wo[0], ln2_s[0].reshape(1, D), ln2_b[0].reshape(1, D),
        W1[0], W2[0],
        ln1_s[1].reshape(1, D), ln1_b[1].reshape(1, D), Wq[1], Wk[1], Wv[1])
    wv1, z1 = edge_attention(q1, kv1, src, dst)
    zp1 = z1.reshape(NC, N_PAD, H)
    out = _post_gen(
        x1, wv1, zp1, Wo[1], ln2_s[1].reshape(1, D), ln2_b[1].reshape(1, D),
        W1[1], W2[1],
        gen_ln_s.reshape(1, D), gen_ln_b.reshape(1, D), Wgen)
    return out


# speedup vs baseline: 20.1534x; 1.0582x over previous
import functools

import jax
import jax.numpy as jnp
from jax import lax
from jax.experimental import pallas as pl
from jax.experimental.pallas import tpu as pltpu
from jax.experimental.pallas import tpu_sc as plsc

N = 10000
E = 320000
D = 128
H = 8
DK = 16
V = 512
P = 64
FF = 512

R = 400
GRID = N // R

NC = 2
NS = 16
NW = NC * NS
EW = E // NW
B = 40
NB = EW // B
N_PAD = 10240
NZ = N_PAD // 16
RC = 8
SCALE = 1.0 / (DK ** 0.5)


def _ln(x, s, b):
    m = jnp.mean(x, axis=-1, keepdims=True)
    var = jnp.mean(x * x, axis=-1, keepdims=True) - m * m
    return (x - m) * jax.lax.rsqrt(var + 1e-5) * s + b


def _qkv(xn, wq_ref, wk_ref, wv_ref, q_out, kv_out):
    q_out[...] = jnp.dot(xn, wq_ref[...], preferred_element_type=jnp.float32)
    kv_out[:, 0:D] = jnp.dot(xn, wk_ref[...], preferred_element_type=jnp.float32)
    kv_out[:, D:2 * D] = jnp.dot(xn, wv_ref[...], preferred_element_type=jnp.float32)


def _post(x, wv_ref, z_ref, wo_ref, ln2s_ref, ln2b_ref, w1_ref, w2_ref):
    wv = wv_ref[0] + wv_ref[1]
    z = z_ref[0] + z_ref[1]
    r = 1.0 / (z + 1e-9)
    col = lax.broadcasted_iota(jnp.int32, (H, D), 1) // DK
    row = lax.broadcasted_iota(jnp.int32, (H, D), 0)
    expand = (col == row).astype(jnp.float32)
    rrep = jnp.dot(r, expand, preferred_element_type=jnp.float32)
    o = jnp.dot(wv * rrep, wo_ref[...], preferred_element_type=jnp.float32)
    x1 = x + o
    xn2 = _ln(x1, ln2s_ref[...], ln2b_ref[...])
    f = jnp.dot(
        jax.nn.relu(jnp.dot(xn2, w1_ref[...], preferred_element_type=jnp.float32)),
        w2_ref[...], preferred_element_type=jnp.float32)
    return x1 + f


def _embed_qkv_body(vals_ref, pos_ref, coord_ref, ptab_ref, vtab_ref,
                    ln1s_ref, ln1b_ref, wq_ref, wk_ref, wv_ref,
                    x_out, q_out, kv_out):
    vals = vals_ref[...]
    pos = pos_ref[...]
    c = pos % 3
    p = pos // 3
    x = jnp.where(c == 0, coord_ref[0:1, :],
                  jnp.where(c == 1, coord_ref[1:2, :], coord_ref[2:3, :]))
    oh_p = (p == lax.broadcasted_iota(jnp.int32, (R, P), 1)).astype(jnp.float32)
    x = x + jnp.dot(oh_p, ptab_ref[...], preferred_element_type=jnp.float32)
    oh_v = (vals == lax.broadcasted_iota(jnp.int32, (R, V), 1)).astype(jnp.float32)
    x = x + jnp.dot(oh_v, vtab_ref[...], preferred_element_type=jnp.float32)
    x_out[...] = x
    xn = _ln(x, ln1s_ref[...], ln1b_ref[...])
    _qkv(xn, wq_ref, wk_ref, wv_ref, q_out, kv_out)


def _post_qkv_body(x_ref, wv_ref, z_ref, wo_ref, ln2s_ref, ln2b_ref,
                   w1_ref, w2_ref, ln1s_ref, ln1b_ref, wq_ref, wk_ref, wv2_ref,
                   x_out, q_out, kv_out):
    x2 = _post(x_ref[...], wv_ref, z_ref, wo_ref, ln2s_ref, ln2b_ref,
               w1_ref, w2_ref)
    x_out[...] = x2
    xn = _ln(x2, ln1s_ref[...], ln1b_ref[...])
    _qkv(xn, wq_ref, wk_ref, wv2_ref, q_out, kv_out)


def _post_gen_body(x_ref, wv_ref, z_ref, wo_ref, ln2s_ref, ln2b_ref,
                   w1_ref, w2_ref, glns_ref, glnb_ref, wgen_ref, out_ref):
    x2 = _post(x_ref[...], wv_ref, z_ref, wo_ref, ln2s_ref, ln2b_ref,
               w1_ref, w2_ref)
    xg = _ln(x2, glns_ref[...], glnb_ref[...])
    logits = jnp.dot(xg, wgen_ref[...], preferred_element_type=jnp.float32)
    m = jnp.max(logits, axis=-1, keepdims=True)
    lse = m + jnp.log(jnp.sum(jnp.exp(logits - m), axis=-1, keepdims=True))
    out_ref[...] = logits - lse


def _full(shape):
    return pl.BlockSpec(shape, lambda i: tuple(0 for _ in shape))


_ROW = pl.BlockSpec((R, D), lambda i: (i, 0))
_ROW_KV = pl.BlockSpec((R, 2 * D), lambda i: (i, 0))
_ROW_WV = pl.BlockSpec((NC, R, D), lambda i: (0, i, 0))
_ROW_Z = pl.BlockSpec((NC, R, H), lambda i: (0, i, 0))
_ROW_IDX = pl.BlockSpec((R, 1), lambda i: (i, 0))

_embed_qkv = pl.pallas_call(
    _embed_qkv_body,
    grid=(GRID,),
    in_specs=[_ROW_IDX, _ROW_IDX, _full((8, D)), _full((P, D)), _full((V, D)),
              _full((1, D)), _full((1, D)),
              _full((D, D)), _full((D, D)), _full((D, D))],
    out_specs=[_ROW, _ROW, _ROW_KV],
    out_shape=[jax.ShapeDtypeStruct((N, D), jnp.float32),
               jax.ShapeDtypeStruct((N, D), jnp.float32),
               jax.ShapeDtypeStruct((N, 2 * D), jnp.float32)],
)

_post_qkv = pl.pallas_call(
    _post_qkv_body,
    grid=(GRID,),
    in_specs=[_ROW, _ROW_WV, _ROW_Z, _full((D, D)), _full((1, D)), _full((1, D)),
              _full((D, FF)), _full((FF, D)),
              _full((1, D)), _full((1, D)),
              _full((D, D)), _full((D, D)), _full((D, D))],
    out_specs=[_ROW, _ROW, _ROW_KV],
    out_shape=[jax.ShapeDtypeStruct((N, D), jnp.float32),
               jax.ShapeDtypeStruct((N, D), jnp.float32),
               jax.ShapeDtypeStruct((N, 2 * D), jnp.float32)],
)

_post_gen = pl.pallas_call(
    _post_gen_body,
    grid=(GRID,),
    in_specs=[_ROW, _ROW_WV, _ROW_Z, _full((D, D)), _full((1, D)), _full((1, D)),
              _full((D, FF)), _full((FF, D)),
              _full((1, D)), _full((1, D)), _full((D, V))],
    out_specs=pl.BlockSpec((R, V), lambda i: (i, 0)),
    out_shape=jax.ShapeDtypeStruct((N, V), jnp.float32),
)


def _edge_body(q_hbm, kv_hbm, src_hbm, dst_hbm, wv_hbm, z_hbm,
               src_idx0, dst_idx0, dst_pad0, zidx0, kvrows0, qrows0,
               src_idx1, dst_idx1, dst_pad1, zidx1, kvrows1, qrows1,
               sdst, szidx, wvrows, zrows, chunk, pb, ztmp,
               acc_wv, acc_z,
               ksem0, qsem0, ksem1, qsem1, isem0, isem1, wsem, zsem):
    cid = lax.axis_index("c")
    sid = lax.axis_index("s")
    wid = cid * NS + sid
    zeros16 = jnp.zeros((16,), jnp.float32)
    lanes = lax.broadcasted_iota(jnp.int32, (16,), 0)

    bufs = ((src_idx0, dst_idx0, dst_pad0, zidx0, kvrows0, qrows0,
             ksem0, qsem0, isem0),
            (src_idx1, dst_idx1, dst_pad1, zidx1, kvrows1, qrows1,
             ksem1, qsem1, isem1))

    def _zero_row(rr, _):
        for cc in range(D // 16):
            chunk[rr, pl.ds(cc * 16, 16)] = zeros16
        return 0
    lax.fori_loop(0, RC, _zero_row, 0)

    def _zero_wv(t, _):
        pltpu.sync_copy(chunk, acc_wv.at[pl.ds(sid * (N_PAD // NS) + t * RC, RC)])
        return 0
    lax.fori_loop(0, N_PAD // NS // RC, _zero_wv, 0)

    def _zero_z(t, _):
        pltpu.sync_copy(chunk, acc_z.at[pl.ds(sid * (NZ // NS) + t * RC, RC)])
        return 0
    lax.fori_loop(0, NZ // NS // RC, _zero_z, 0)
    ztmp[pl.ds(0, 16)] = zeros16
    ztmp[pl.ds(16, 16)] = zeros16
    plsc.subcore_barrier()

    def _issue_idx(b, j):
        base = wid * EW + j * B
        pltpu.async_copy(src_hbm.at[pl.ds(base, B)], bufs[b][0], bufs[b][8])
        pltpu.async_copy(dst_hbm.at[pl.ds(base, B)], bufs[b][1], bufs[b][8])

    def _wait_idx(b):
        pltpu.make_async_copy(src_hbm.at[pl.ds(0, B)], bufs[b][0], bufs[b][8]).wait()
        pltpu.make_async_copy(dst_hbm.at[pl.ds(0, B)], bufs[b][1], bufs[b][8]).wait()

    def _issue_gather(b):
        pltpu.async_copy(kv_hbm.at[bufs[b][0]], bufs[b][4], bufs[b][6])
        pltpu.async_copy(q_hbm.at[bufs[b][1]], bufs[b][5], bufs[b][7])

    def _wait_gather(b):
        pltpu.make_async_copy(kv_hbm.at[bufs[b][0]], bufs[b][4], bufs[b][6]).wait()
        pltpu.make_async_copy(q_hbm.at[bufs[b][1]], bufs[b][5], bufs[b][7]).wait()

    def _wait_scatter():
        pltpu.make_async_copy(wvrows, acc_wv.at[sdst], wsem).wait()
        pltpu.make_async_copy(zrows, acc_z.at[szidx], zsem).wait()

    def _compute_scatter(b, j):
        src_b, dst_b, pad_b, zix_b, kvrows_b, qrows_b = bufs[b][:6]
        for c in range(-(-B // 16)):
            cc = min(c * 16, B - 16)
            dv = dst_b[pl.ds(cc, 16)]
            sdst[pl.ds(cc, 16)] = dv
            szidx[pl.ds(cc, 16)] = lax.shift_right_logical(dv, 4)
            pad_b[pl.ds(cc, 16)] = dv
        pad_b[pl.ds(B, 16)] = jnp.zeros((16,), jnp.int32)

        def _zrow_zero(e, _):
            for cc in range(D // 16):
                zrows[e, pl.ds(cc * 16, 16)] = zeros16
            return 0
        lax.fori_loop(0, B, _zrow_zero, 0)

        def _edge(e, _):
            d = pad_b[pl.ds(e, 16)][0]
            svec = zeros16
            for h in range(H):
                kvec = kvrows_b[e, pl.ds(h * DK, 16)]
                qvec = qrows_b[e, pl.ds(h * DK, 16)]
                r = kvec * qvec
                for step, off in ((0, 8), (16, 4), (32, 2), (48, 1)):
                    pb[pl.ds(step, 16)] = r
                    r = r + pb[pl.ds(step + off, 16)]
                svec = jnp.where(lanes == h, r[0], svec)
            scv = jnp.exp(jnp.clip(svec * SCALE, -10.0, 10.0))
            zrow = jnp.where(lanes < H, scv, 0.0)
            ztmp[pl.ds(8, 16)] = zrow
            zrow_hi = ztmp[pl.ds(0, 16)]
            o = jnp.bitwise_and(d, 15) * H
            hi = o > D - 16
            off = jnp.where(hi, D - 16, o)
            hf = jnp.where(hi, 1.0, 0.0)
            zrows[e, pl.ds(off, 16)] = zrow + (zrow_hi - zrow) * hf
            for h in range(H):
                sch = scv[h]
                vvec = kvrows_b[e, pl.ds(D + h * DK, 16)]
                wvrows[e, pl.ds(h * DK, 16)] = vvec * jnp.broadcast_to(sch, (16,))
            return 0
        lax.fori_loop(0, B, _edge, 0)
        pltpu.async_copy(wvrows, acc_wv.at[sdst], wsem, add=True)
        pltpu.async_copy(zrows, acc_z.at[szidx], zsem, add=True)

    _issue_idx(0, 0)
    _wait_idx(0)
    _issue_gather(0)
    _issue_idx(1, 1)

    def _super(g, _):
        for b in range(2):
            j = 2 * g + b
            @pl.when(j + 1 < NB)
            def _():
                _wait_idx(1 - b)
                _issue_gather(1 - b)
            _wait_gather(b)
            @pl.when(j + 2 < NB)
            def _():
                _issue_idx(b, j + 2)
            @pl.when(j >= 1)
            def _():
                _wait_scatter()
            _compute_scatter(b, j)
        return 0

    lax.fori_loop(0, NB // 2, _super, 0)
    _wait_scatter()
    plsc.subcore_barrier()

    def _dump_wv(t, _):
        r0 = sid * (N_PAD // NS) + t * RC
        pltpu.sync_copy(acc_wv.at[pl.ds(r0, RC)], chunk)
        pltpu.sync_copy(chunk, wv_hbm.at[cid, pl.ds(r0, RC)])
        return 0
    lax.fori_loop(0, N_PAD // NS // RC, _dump_wv, 0)

    def _dump_z(t, _):
        r0 = sid * (NZ // NS) + t * RC
        pltpu.sync_copy(acc_z.at[pl.ds(r0, RC)], chunk)
        pltpu.sync_copy(chunk, z_hbm.at[cid, pl.ds(r0, RC)])
        return 0
    lax.fori_loop(0, NZ // NS // RC, _dump_z, 0)


@functools.lru_cache(maxsize=1)
def _edge_attention():
    return pl.kernel(
        _edge_body,
        out_type=(jax.ShapeDtypeStruct((NC, N_PAD, D), jnp.float32),
                  jax.ShapeDtypeStruct((NC, NZ, D), jnp.float32)),
        mesh=plsc.VectorSubcoreMesh(core_axis_name="c", subcore_axis_name="s",
                                    num_cores=NC, num_subcores=NS),
        scratch_types=(
            [pltpu.VMEM((B,), jnp.int32),
             pltpu.VMEM((B,), jnp.int32),
             pltpu.VMEM((B + 16,), jnp.int32),
             pltpu.VMEM((B,), jnp.int32),
             pltpu.VMEM((B, 2 * D), jnp.float32),
             pltpu.VMEM((B, D), jnp.float32)] * 2
            + [
                pltpu.VMEM((B,), jnp.int32),
                pltpu.VMEM((B,), jnp.int32),
                pltpu.VMEM((B, D), jnp.float32),
                pltpu.VMEM((B, D), jnp.float32),
                pltpu.VMEM((RC, D), jnp.float32),
                pltpu.VMEM((80,), jnp.float32),
                pltpu.VMEM((32,), jnp.float32),
                pltpu.VMEM_SHARED((N_PAD, D), jnp.float32),
                pltpu.VMEM_SHARED((NZ, D), jnp.float32),
                pltpu.SemaphoreType.DMA,
                pltpu.SemaphoreType.DMA,
                pltpu.SemaphoreType.DMA,
                pltpu.SemaphoreType.DMA,
                pltpu.SemaphoreType.DMA,
                pltpu.SemaphoreType.DMA,
                pltpu.SemaphoreType.DMA,
                pltpu.SemaphoreType.DMA,
            ]),
    )


def kernel(tgt_values, tgt_positions, edge_src, edge_dst, coord_tab, pos_tab,
           val_tab, ln1_s, ln1_b, Wq, Wk, Wv, Wo, ln2_s, ln2_b, W1, W2,
           gen_ln_s, gen_ln_b, Wgen):
    vals2 = tgt_values.astype(jnp.int32).reshape(N, 1)
    pos2 = tgt_positions.astype(jnp.int32).reshape(N, 1)
    src = edge_src.astype(jnp.int32)
    dst = edge_dst.astype(jnp.int32)
    coordp = jnp.zeros((8, D), jnp.float32).at[0:3].set(coord_tab)

    x0, q0, kv0 = _embed_qkv(
        vals2, pos2, coordp, pos_tab, val_tab,
        ln1_s[0].reshape(1, D), ln1_b[0].reshape(1, D), Wq[0], Wk[0], Wv[0])
    edge_attention = _edge_attention()
    wv0, z0 = edge_attention(q0, kv0, src, dst)
    zp0 = z0.reshape(NC, N_PAD, H)
    x1, q1, kv1 = _post_qkv(
        x0, wv0, zp0, Wo[0], ln2_s[0].reshape(1, D), ln2_b[0].reshape(1, D),
        W1[0], W2[0],
        ln1_s[1].reshape(1, D), ln1_b[1].reshape(1, D), Wq[1], Wk[1], Wv[1])
    wv1, z1 = edge_attention(q1, kv1, src, dst)
    zp1 = z1.reshape(NC, N_PAD, H)
    out = _post_gen(
        x1, wv1, zp1, Wo[1], ln2_s[1].reshape(1, D), ln2_b[1].reshape(1, D),
        W1[1], W2[1],
        gen_ln_s.reshape(1, D), gen_ln_b.reshape(1, D), Wgen)
    return out


# speedup vs baseline: 20.1593x; 1.0003x over previous
import functools

import jax
import jax.numpy as jnp
from jax import lax
from jax.experimental import pallas as pl
from jax.experimental.pallas import tpu as pltpu
from jax.experimental.pallas import tpu_sc as plsc

N = 10000
E = 320000
D = 128
H = 8
DK = 16
V = 512
P = 64
FF = 512

R = 400
GRID = N // R

NC = 2
NS = 16
NW = NC * NS
EW = E // NW
B = 40
NB = EW // B
N_PAD = 10240
NZ = N_PAD // 16
RC = 8
SCALE = 1.0 / (DK ** 0.5)


def _ln(x, s, b):
    m = jnp.mean(x, axis=-1, keepdims=True)
    var = jnp.mean(x * x, axis=-1, keepdims=True) - m * m
    return (x - m) * jax.lax.rsqrt(var + 1e-5) * s + b


def _qkv(xn, wq_ref, wk_ref, wv_ref, q_out, kv_out):
    q_out[...] = jnp.dot(xn, wq_ref[...], preferred_element_type=jnp.float32)
    kv_out[:, 0:D] = jnp.dot(xn, wk_ref[...], preferred_element_type=jnp.float32)
    kv_out[:, D:2 * D] = jnp.dot(xn, wv_ref[...], preferred_element_type=jnp.float32)


def _post(x, wv_ref, z_ref, wo_ref, ln2s_ref, ln2b_ref, w1_ref, w2_ref):
    wv = wv_ref[0] + wv_ref[1]
    z = z_ref[0] + z_ref[1]
    r = 1.0 / (z + 1e-9)
    col = lax.broadcasted_iota(jnp.int32, (H, D), 1) // DK
    row = lax.broadcasted_iota(jnp.int32, (H, D), 0)
    expand = (col == row).astype(jnp.float32)
    rrep = jnp.dot(r, expand, preferred_element_type=jnp.float32)
    o = jnp.dot(wv * rrep, wo_ref[...], preferred_element_type=jnp.float32)
    x1 = x + o
    xn2 = _ln(x1, ln2s_ref[...], ln2b_ref[...])
    f = jnp.dot(
        jax.nn.relu(jnp.dot(xn2, w1_ref[...], preferred_element_type=jnp.float32)),
        w2_ref[...], preferred_element_type=jnp.float32)
    return x1 + f


def _embed_qkv_body(vals_ref, pos_ref, coord_ref, ptab_ref, vtab_ref,
                    ln1s_ref, ln1b_ref, wq_ref, wk_ref, wv_ref,
                    x_out, q_out, kv_out):
    vals = vals_ref[...]
    pos = pos_ref[...]
    c = pos % 3
    p = pos // 3
    x = jnp.where(c == 0, coord_ref[0:1, :],
                  jnp.where(c == 1, coord_ref[1:2, :], coord_ref[2:3, :]))
    oh_p = (p == lax.broadcasted_iota(jnp.int32, (R, P), 1)).astype(jnp.float32)
    x = x + jnp.dot(oh_p, ptab_ref[...], preferred_element_type=jnp.float32)
    oh_v = (vals == lax.broadcasted_iota(jnp.int32, (R, V), 1)).astype(jnp.float32)
    x = x + jnp.dot(oh_v, vtab_ref[...], preferred_element_type=jnp.float32)
    x_out[...] = x
    xn = _ln(x, ln1s_ref[...], ln1b_ref[...])
    _qkv(xn, wq_ref, wk_ref, wv_ref, q_out, kv_out)


def _post_qkv_body(x_ref, wv_ref, z_ref, wo_ref, ln2s_ref, ln2b_ref,
                   w1_ref, w2_ref, ln1s_ref, ln1b_ref, wq_ref, wk_ref, wv2_ref,
                   x_out, q_out, kv_out):
    x2 = _post(x_ref[...], wv_ref, z_ref, wo_ref, ln2s_ref, ln2b_ref,
               w1_ref, w2_ref)
    x_out[...] = x2
    xn = _ln(x2, ln1s_ref[...], ln1b_ref[...])
    _qkv(xn, wq_ref, wk_ref, wv2_ref, q_out, kv_out)


def _post_gen_body(x_ref, wv_ref, z_ref, wo_ref, ln2s_ref, ln2b_ref,
                   w1_ref, w2_ref, glns_ref, glnb_ref, wgen_ref, out_ref):
    x2 = _post(x_ref[...], wv_ref, z_ref, wo_ref, ln2s_ref, ln2b_ref,
               w1_ref, w2_ref)
    xg = _ln(x2, glns_ref[...], glnb_ref[...])
    logits = jnp.dot(xg, wgen_ref[...], preferred_element_type=jnp.float32)
    m = jnp.max(logits, axis=-1, keepdims=True)
    lse = m + jnp.log(jnp.sum(jnp.exp(logits - m), axis=-1, keepdims=True))
    out_ref[...] = logits - lse


def _full(shape):
    return pl.BlockSpec(shape, lambda i: tuple(0 for _ in shape))


_ROW = pl.BlockSpec((R, D), lambda i: (i, 0))
_ROW_KV = pl.BlockSpec((R, 2 * D), lambda i: (i, 0))
_ROW_WV = pl.BlockSpec((NC, R, D), lambda i: (0, i, 0))
_ROW_Z = pl.BlockSpec((NC, R, H), lambda i: (0, i, 0))
_ROW_IDX = pl.BlockSpec((R, 1), lambda i: (i, 0))

_embed_qkv = pl.pallas_call(
    _embed_qkv_body,
    grid=(GRID,),
    in_specs=[_ROW_IDX, _ROW_IDX, _full((8, D)), _full((P, D)), _full((V, D)),
              _full((1, D)), _full((1, D)),
              _full((D, D)), _full((D, D)), _full((D, D))],
    out_specs=[_ROW, _ROW, _ROW_KV],
    out_shape=[jax.ShapeDtypeStruct((N, D), jnp.float32),
               jax.ShapeDtypeStruct((N, D), jnp.float32),
               jax.ShapeDtypeStruct((N, 2 * D), jnp.float32)],
)

_post_qkv = pl.pallas_call(
    _post_qkv_body,
    grid=(GRID,),
    in_specs=[_ROW, _ROW_WV, _ROW_Z, _full((D, D)), _full((1, D)), _full((1, D)),
              _full((D, FF)), _full((FF, D)),
              _full((1, D)), _full((1, D)),
              _full((D, D)), _full((D, D)), _full((D, D))],
    out_specs=[_ROW, _ROW, _ROW_KV],
    out_shape=[jax.ShapeDtypeStruct((N, D), jnp.float32),
               jax.ShapeDtypeStruct((N, D), jnp.float32),
               jax.ShapeDtypeStruct((N, 2 * D), jnp.float32)],
)

_post_gen = pl.pallas_call(
    _post_gen_body,
    grid=(GRID,),
    in_specs=[_ROW, _ROW_WV, _ROW_Z, _full((D, D)), _full((1, D)), _full((1, D)),
              _full((D, FF)), _full((FF, D)),
              _full((1, D)), _full((1, D)), _full((D, V))],
    out_specs=pl.BlockSpec((R, V), lambda i: (i, 0)),
    out_shape=jax.ShapeDtypeStruct((N, V), jnp.float32),
)


def _edge_body(q_hbm, kv_hbm, src_hbm, dst_hbm, wv_hbm, z_hbm,
               src_idx0, dst_idx0, dst_pad0, zidx0, kvrows0, qrows0,
               src_idx1, dst_idx1, dst_pad1, zidx1, kvrows1, qrows1,
               sdst, szidx, wvrows, zrows, chunk, pb, ztmp,
               acc_wv, acc_z,
               ksem0, qsem0, ksem1, qsem1, isem0, isem1, wsem, zsem):
    cid = lax.axis_index("c")
    sid = lax.axis_index("s")
    wid = cid * NS + sid
    zeros16 = jnp.zeros((16,), jnp.float32)
    lanes = lax.broadcasted_iota(jnp.int32, (16,), 0)

    bufs = ((src_idx0, dst_idx0, dst_pad0, zidx0, kvrows0, qrows0,
             ksem0, qsem0, isem0),
            (src_idx1, dst_idx1, dst_pad1, zidx1, kvrows1, qrows1,
             ksem1, qsem1, isem1))

    def _zero_row(rr, _):
        for cc in range(D // 16):
            chunk[rr, pl.ds(cc * 16, 16)] = zeros16
        return 0
    lax.fori_loop(0, RC, _zero_row, 0)

    def _zero_wv(t, _):
        pltpu.sync_copy(chunk, acc_wv.at[pl.ds(sid * (N_PAD // NS) + t * RC, RC)])
        return 0
    lax.fori_loop(0, N_PAD // NS // RC, _zero_wv, 0)

    def _zero_z(t, _):
        pltpu.sync_copy(chunk, acc_z.at[pl.ds(sid * (NZ // NS) + t * RC, RC)])
        return 0
    lax.fori_loop(0, NZ // NS // RC, _zero_z, 0)
    ztmp[pl.ds(0, 16)] = zeros16
    ztmp[pl.ds(16, 16)] = zeros16
    plsc.subcore_barrier()

    def _issue_idx(b, j):
        base = wid * EW + j * B
        pltpu.async_copy(src_hbm.at[pl.ds(base, B)], bufs[b][0], bufs[b][8])
        pltpu.async_copy(dst_hbm.at[pl.ds(base, B)], bufs[b][1], bufs[b][8])

    def _wait_idx(b):
        pltpu.make_async_copy(src_hbm.at[pl.ds(0, B)], bufs[b][0], bufs[b][8]).wait()
        pltpu.make_async_copy(dst_hbm.at[pl.ds(0, B)], bufs[b][1], bufs[b][8]).wait()

    def _issue_gather(b):
        pltpu.async_copy(kv_hbm.at[bufs[b][0]], bufs[b][4], bufs[b][6])
        pltpu.async_copy(q_hbm.at[bufs[b][1]], bufs[b][5], bufs[b][7])

    def _wait_gather(b):
        pltpu.make_async_copy(kv_hbm.at[bufs[b][0]], bufs[b][4], bufs[b][6]).wait()
        pltpu.make_async_copy(q_hbm.at[bufs[b][1]], bufs[b][5], bufs[b][7]).wait()

    def _wait_scatter():
        pltpu.make_async_copy(wvrows, acc_wv.at[sdst], wsem).wait()
        pltpu.make_async_copy(zrows, acc_z.at[szidx], zsem).wait()

    def _derive(b):
        dst_b, pad_b = bufs[b][1], bufs[b][2]
        for c in range(-(-B // 16)):
            cc = min(c * 16, B - 16)
            dv = dst_b[pl.ds(cc, 16)]
            sdst[pl.ds(cc, 16)] = dv
            szidx[pl.ds(cc, 16)] = lax.shift_right_logical(dv, 4)
            pad_b[pl.ds(cc, 16)] = dv
        pad_b[pl.ds(B, 16)] = jnp.zeros((16,), jnp.int32)

    def _compute_scatter(b, j):
        pad_b, kvrows_b, qrows_b = bufs[b][2], bufs[b][4], bufs[b][5]

        def _zrow_zero(e, _):
            for cc in range(D // 16):
                zrows[e, pl.ds(cc * 16, 16)] = zeros16
            return 0
        lax.fori_loop(0, B, _zrow_zero, 0)

        def _edge(e, _):
            d = pad_b[pl.ds(e, 16)][0]
            svec = zeros16
            for h in range(H):
                kvec = kvrows_b[e, pl.ds(h * DK, 16)]
                qvec = qrows_b[e, pl.ds(h * DK, 16)]
                r = kvec * qvec
                for step, off in ((0, 8), (16, 4), (32, 2), (48, 1)):
                    pb[pl.ds(step, 16)] = r
                    r = r + pb[pl.ds(step + off, 16)]
                svec = jnp.where(lanes == h, r[0], svec)
            scv = jnp.exp(jnp.clip(svec * SCALE, -10.0, 10.0))
            zrow = jnp.where(lanes < H, scv, 0.0)
            ztmp[pl.ds(8, 16)] = zrow
            zrow_hi = ztmp[pl.ds(0, 16)]
            o = jnp.bitwise_and(d, 15) * H
            hi = o > D - 16
            off = jnp.where(hi, D - 16, o)
            hf = jnp.where(hi, 1.0, 0.0)
            zrows[e, pl.ds(off, 16)] = zrow + (zrow_hi - zrow) * hf
            for h in range(H):
                sch = scv[h]
                vvec = kvrows_b[e, pl.ds(D + h * DK, 16)]
                wvrows[e, pl.ds(h * DK, 16)] = vvec * jnp.broadcast_to(sch, (16,))
            return 0
        lax.fori_loop(0, B, _edge, 0)
        pltpu.async_copy(wvrows, acc_wv.at[sdst], wsem, add=True)
        pltpu.async_copy(zrows, acc_z.at[szidx], zsem, add=True)

    _issue_idx(0, 0)
    _wait_idx(0)
    _issue_gather(0)
    _issue_idx(1, 1)

    def _super(g, _):
        for b in range(2):
            j = 2 * g + b
            @pl.when(j + 1 < NB)
            def _():
                _wait_idx(1 - b)
                _issue_gather(1 - b)
            _wait_gather(b)
            @pl.when(j >= 1)
            def _():
                _wait_scatter()
            _derive(b)
            @pl.when(j + 2 < NB)
            def _():
                _issue_idx(b, j + 2)
            _compute_scatter(b, j)
        return 0

    lax.fori_loop(0, NB // 2, _super, 0)
    _wait_scatter()
    plsc.subcore_barrier()

    def _dump_wv(t, _):
        r0 = sid * (N_PAD // NS) + t * RC
        pltpu.sync_copy(acc_wv.at[pl.ds(r0, RC)], chunk)
        pltpu.sync_copy(chunk, wv_hbm.at[cid, pl.ds(r0, RC)])
        return 0
    lax.fori_loop(0, N_PAD // NS // RC, _dump_wv, 0)

    def _dump_z(t, _):
        r0 = sid * (NZ // NS) + t * RC
        pltpu.sync_copy(acc_z.at[pl.ds(r0, RC)], chunk)
        pltpu.sync_copy(chunk, z_hbm.at[cid, pl.ds(r0, RC)])
        return 0
    lax.fori_loop(0, NZ // NS // RC, _dump_z, 0)


@functools.lru_cache(maxsize=1)
def _edge_attention():
    return pl.kernel(
        _edge_body,
        out_type=(jax.ShapeDtypeStruct((NC, N_PAD, D), jnp.float32),
                  jax.ShapeDtypeStruct((NC, NZ, D), jnp.float32)),
        mesh=plsc.VectorSubcoreMesh(core_axis_name="c", subcore_axis_name="s",
                                    num_cores=NC, num_subcores=NS),
        scratch_types=(
            [pltpu.VMEM((B,), jnp.int32),
             pltpu.VMEM((B,), jnp.int32),
             pltpu.VMEM((B + 16,), jnp.int32),
             pltpu.VMEM((B,), jnp.int32),
             pltpu.VMEM((B, 2 * D), jnp.float32),
             pltpu.VMEM((B, D), jnp.float32)] * 2
            + [
                pltpu.VMEM((B,), jnp.int32),
                pltpu.VMEM((B,), jnp.int32),
                pltpu.VMEM((B, D), jnp.float32),
                pltpu.VMEM((B, D), jnp.float32),
                pltpu.VMEM((RC, D), jnp.float32),
                pltpu.VMEM((80,), jnp.float32),
                pltpu.VMEM((32,), jnp.float32),
                pltpu.VMEM_SHARED((N_PAD, D), jnp.float32),
                pltpu.VMEM_SHARED((NZ, D), jnp.float32),
                pltpu.SemaphoreType.DMA,
                pltpu.SemaphoreType.DMA,
                pltpu.SemaphoreType.DMA,
                pltpu.SemaphoreType.DMA,
                pltpu.SemaphoreType.DMA,
                pltpu.SemaphoreType.DMA,
                pltpu.SemaphoreType.DMA,
                pltpu.SemaphoreType.DMA,
            ]),
    )


def kernel(tgt_values, tgt_positions, edge_src, edge_dst, coord_tab, pos_tab,
           val_tab, ln1_s, ln1_b, Wq, Wk, Wv, Wo, ln2_s, ln2_b, W1, W2,
           gen_ln_s, gen_ln_b, Wgen):
    vals2 = tgt_values.astype(jnp.int32).reshape(N, 1)
    pos2 = tgt_positions.astype(jnp.int32).reshape(N, 1)
    src = edge_src.astype(jnp.int32)
    dst = edge_dst.astype(jnp.int32)
    coordp = jnp.zeros((8, D), jnp.float32).at[0:3].set(coord_tab)

    x0, q0, kv0 = _embed_qkv(
        vals2, pos2, coordp, pos_tab, val_tab,
        ln1_s[0].reshape(1, D), ln1_b[0].reshape(1, D), Wq[0], Wk[0], Wv[0])
    edge_attention = _edge_attention()
    wv0, z0 = edge_attention(q0, kv0, src, dst)
    zp0 = z0.reshape(NC, N_PAD, H)
    x1, q1, kv1 = _post_qkv(
        x0, wv0, zp0, Wo[0], ln2_s[0].reshape(1, D), ln2_b[0].reshape(1, D),
        W1[0], W2[0],
        ln1_s[1].reshape(1, D), ln1_b[1].reshape(1, D), Wq[1], Wk[1], Wv[1])
    wv1, z1 = edge_attention(q1, kv1, src, dst)
    zp1 = z1.reshape(NC, N_PAD, H)
    out = _post_gen(
        x1, wv1, zp1, Wo[1], ln2_s[1].reshape(1, D), ln2_b[1].reshape(1, D),
        W1[1], W2[1],
        gen_ln_s.reshape(1, D), gen_ln_b.reshape(1, D), Wgen)
    return out


# speedup vs baseline: 20.1612x; 1.0001x over previous
import functools

import jax
import jax.numpy as jnp
from jax import lax
from jax.experimental import pallas as pl
from jax.experimental.pallas import tpu as pltpu
from jax.experimental.pallas import tpu_sc as plsc

N = 10000
E = 320000
D = 128
H = 8
DK = 16
V = 512
P = 64
FF = 512

R = 400
GRID = N // R

NC = 2
NS = 16
NW = NC * NS
EW = E // NW
B = 40
NB = EW // B
N_PAD = 10240
NZ = N_PAD // 16
RC = 8
SCALE = 1.0 / (DK ** 0.5)


def _ln(x, s, b):
    m = jnp.mean(x, axis=-1, keepdims=True)
    var = jnp.mean(x * x, axis=-1, keepdims=True) - m * m
    return (x - m) * jax.lax.rsqrt(var + 1e-5) * s + b


def _qkv(xn, wq_ref, wk_ref, wv_ref, q_out, kv_out):
    q_out[...] = jnp.dot(xn, wq_ref[...], preferred_element_type=jnp.float32)
    kv_out[:, 0:D] = jnp.dot(xn, wk_ref[...], preferred_element_type=jnp.float32)
    kv_out[:, D:2 * D] = jnp.dot(xn, wv_ref[...], preferred_element_type=jnp.float32)


def _post(x, wv_ref, z_ref, wo_ref, ln2s_ref, ln2b_ref, w1_ref, w2_ref):
    wv = wv_ref[0] + wv_ref[1]
    z = z_ref[0] + z_ref[1]
    r = 1.0 / (z + 1e-9)
    col = lax.broadcasted_iota(jnp.int32, (H, D), 1) // DK
    row = lax.broadcasted_iota(jnp.int32, (H, D), 0)
    expand = (col == row).astype(jnp.float32)
    rrep = jnp.dot(r, expand, preferred_element_type=jnp.float32)
    o = jnp.dot(wv * rrep, wo_ref[...], preferred_element_type=jnp.float32)
    x1 = x + o
    xn2 = _ln(x1, ln2s_ref[...], ln2b_ref[...])
    f = jnp.dot(
        jax.nn.relu(jnp.dot(xn2, w1_ref[...], preferred_element_type=jnp.float32)),
        w2_ref[...], preferred_element_type=jnp.float32)
    return x1 + f


def _embed_qkv_body(vals_ref, pos_ref, coord_ref, ptab_ref, vtab_ref,
                    ln1s_ref, ln1b_ref, wq_ref, wk_ref, wv_ref,
                    x_out, q_out, kv_out):
    vals = vals_ref[...]
    pos = pos_ref[...]
    c = pos % 3
    p = pos // 3
    x = jnp.where(c == 0, coord_ref[0:1, :],
                  jnp.where(c == 1, coord_ref[1:2, :], coord_ref[2:3, :]))
    oh_p = (p == lax.broadcasted_iota(jnp.int32, (R, P), 1)).astype(jnp.float32)
    x = x + jnp.dot(oh_p, ptab_ref[...], preferred_element_type=jnp.float32)
    oh_v = (vals == lax.broadcasted_iota(jnp.int32, (R, V), 1)).astype(jnp.float32)
    x = x + jnp.dot(oh_v, vtab_ref[...], preferred_element_type=jnp.float32)
    x_out[...] = x
    xn = _ln(x, ln1s_ref[...], ln1b_ref[...])
    _qkv(xn, wq_ref, wk_ref, wv_ref, q_out, kv_out)


def _post_qkv_body(x_ref, wv_ref, z_ref, wo_ref, ln2s_ref, ln2b_ref,
                   w1_ref, w2_ref, ln1s_ref, ln1b_ref, wq_ref, wk_ref, wv2_ref,
                   x_out, q_out, kv_out):
    x2 = _post(x_ref[...], wv_ref, z_ref, wo_ref, ln2s_ref, ln2b_ref,
               w1_ref, w2_ref)
    x_out[...] = x2
    xn = _ln(x2, ln1s_ref[...], ln1b_ref[...])
    _qkv(xn, wq_ref, wk_ref, wv2_ref, q_out, kv_out)


def _post_gen_body(x_ref, wv_ref, z_ref, wo_ref, ln2s_ref, ln2b_ref,
                   w1_ref, w2_ref, glns_ref, glnb_ref, wgen_ref, out_ref):
    x2 = _post(x_ref[...], wv_ref, z_ref, wo_ref, ln2s_ref, ln2b_ref,
               w1_ref, w2_ref)
    xg = _ln(x2, glns_ref[...], glnb_ref[...])
    logits = jnp.dot(xg, wgen_ref[...], preferred_element_type=jnp.float32)
    m = jnp.max(logits, axis=-1, keepdims=True)
    lse = m + jnp.log(jnp.sum(jnp.exp(logits - m), axis=-1, keepdims=True))
    out_ref[...] = logits - lse


def _full(shape):
    return pl.BlockSpec(shape, lambda i: tuple(0 for _ in shape))


_ROW = pl.BlockSpec((R, D), lambda i: (i, 0))
_ROW_KV = pl.BlockSpec((R, 2 * D), lambda i: (i, 0))
_ROW_WV = pl.BlockSpec((NC, R, D), lambda i: (0, i, 0))
_ROW_Z = pl.BlockSpec((NC, R, H), lambda i: (0, i, 0))
_ROW_IDX = pl.BlockSpec((R, 1), lambda i: (i, 0))

_embed_qkv = pl.pallas_call(
    _embed_qkv_body,
    grid=(GRID,),
    in_specs=[_ROW_IDX, _ROW_IDX, _full((8, D)), _full((P, D)), _full((V, D)),
              _full((1, D)), _full((1, D)),
              _full((D, D)), _full((D, D)), _full((D, D))],
    out_specs=[_ROW, _ROW, _ROW_KV],
    out_shape=[jax.ShapeDtypeStruct((N, D), jnp.float32),
               jax.ShapeDtypeStruct((N, D), jnp.float32),
               jax.ShapeDtypeStruct((N, 2 * D), jnp.float32)],
)

_post_qkv = pl.pallas_call(
    _post_qkv_body,
    grid=(GRID,),
    in_specs=[_ROW, _ROW_WV, _ROW_Z, _full((D, D)), _full((1, D)), _full((1, D)),
              _full((D, FF)), _full((FF, D)),
              _full((1, D)), _full((1, D)),
              _full((D, D)), _full((D, D)), _full((D, D))],
    out_specs=[_ROW, _ROW, _ROW_KV],
    out_shape=[jax.ShapeDtypeStruct((N, D), jnp.float32),
               jax.ShapeDtypeStruct((N, D), jnp.float32),
               jax.ShapeDtypeStruct((N, 2 * D), jnp.float32)],
)

_post_gen = pl.pallas_call(
    _post_gen_body,
    grid=(GRID,),
    in_specs=[_ROW, _ROW_WV, _ROW_Z, _full((D, D)), _full((1, D)), _full((1, D)),
              _full((D, FF)), _full((FF, D)),
              _full((1, D)), _full((1, D)), _full((D, V))],
    out_specs=pl.BlockSpec((R, V), lambda i: (i, 0)),
    out_shape=jax.ShapeDtypeStruct((N, V), jnp.float32),
)


def _edge_body(q_hbm, kv_hbm, src_hbm, dst_hbm, wv_hbm, z_hbm,
               src_idx0, dst_idx0, dst_pad0, zidx0, kvrows0, qrows0,
               src_idx1, dst_idx1, dst_pad1, zidx1, kvrows1, qrows1,
               sdst, szidx, wvrows, zrows, chunk, pb, ztmp,
               acc_wv, acc_z,
               ksem0, qsem0, ksem1, qsem1, isem0, isem1, wsem, zsem):
    cid = lax.axis_index("c")
    sid = lax.axis_index("s")
    wid = cid * NS + sid
    zeros16 = jnp.zeros((16,), jnp.float32)
    lanes = lax.broadcasted_iota(jnp.int32, (16,), 0)

    bufs = ((src_idx0, dst_idx0, dst_pad0, zidx0, kvrows0, qrows0,
             ksem0, qsem0, isem0),
            (src_idx1, dst_idx1, dst_pad1, zidx1, kvrows1, qrows1,
             ksem1, qsem1, isem1))

    def _zero_row(rr, _):
        for cc in range(D // 16):
            chunk[rr, pl.ds(cc * 16, 16)] = zeros16
        return 0
    lax.fori_loop(0, RC, _zero_row, 0)

    def _zero_wv(t, _):
        pltpu.sync_copy(chunk, acc_wv.at[pl.ds(sid * (N_PAD // NS) + t * RC, RC)])
        return 0
    lax.fori_loop(0, N_PAD // NS // RC, _zero_wv, 0)

    def _zero_z(t, _):
        pltpu.sync_copy(chunk, acc_z.at[pl.ds(sid * (NZ // NS) + t * RC, RC)])
        return 0
    lax.fori_loop(0, NZ // NS // RC, _zero_z, 0)
    ztmp[pl.ds(0, 16)] = zeros16
    ztmp[pl.ds(16, 16)] = zeros16
    plsc.subcore_barrier()

    def _issue_idx(b, j):
        base = wid * EW + j * B
        pltpu.async_copy(src_hbm.at[pl.ds(base, B)], bufs[b][0], bufs[b][8])
        pltpu.async_copy(dst_hbm.at[pl.ds(base, B)], bufs[b][1], bufs[b][8])

    def _wait_idx(b):
        pltpu.make_async_copy(src_hbm.at[pl.ds(0, B)], bufs[b][0], bufs[b][8]).wait()
        pltpu.make_async_copy(dst_hbm.at[pl.ds(0, B)], bufs[b][1], bufs[b][8]).wait()

    def _issue_gather(b):
        pltpu.async_copy(kv_hbm.at[bufs[b][0]], bufs[b][4], bufs[b][6])
        pltpu.async_copy(q_hbm.at[bufs[b][1]], bufs[b][5], bufs[b][7])

    def _wait_gather(b):
        pltpu.make_async_copy(kv_hbm.at[bufs[b][0]], bufs[b][4], bufs[b][6]).wait()
        pltpu.make_async_copy(q_hbm.at[bufs[b][1]], bufs[b][5], bufs[b][7]).wait()

    def _wait_scatter():
        pltpu.make_async_copy(wvrows, acc_wv.at[sdst], wsem).wait()
        pltpu.make_async_copy(zrows, acc_z.at[szidx], zsem).wait()

    def _derive(b):
        dst_b, pad_b = bufs[b][1], bufs[b][2]
        for c in range(-(-B // 16)):
            cc = min(c * 16, B - 16)
            dv = dst_b[pl.ds(cc, 16)]
            sdst[pl.ds(cc, 16)] = dv
            szidx[pl.ds(cc, 16)] = lax.shift_right_logical(dv, 4)
            pad_b[pl.ds(cc, 16)] = dv
        pad_b[pl.ds(B, 16)] = jnp.zeros((16,), jnp.int32)

    def _compute_scatter(b, j):
        pad_b, kvrows_b, qrows_b = bufs[b][2], bufs[b][4], bufs[b][5]

        def _zrow_zero(e, _):
            for cc in range(D // 16):
                zrows[e, pl.ds(cc * 16, 16)] = zeros16
            return 0
        lax.fori_loop(0, B, _zrow_zero, 0)

        def _edge(e, _):
            d = pad_b[pl.ds(e, 16)][0]
            svec = zeros16
            for h in range(H):
                kvec = kvrows_b[e, pl.ds(h * DK, 16)]
                qvec = qrows_b[e, pl.ds(h * DK, 16)]
                r = kvec * qvec
                for step, off in ((0, 8), (16, 4), (32, 2), (48, 1)):
                    pb[pl.ds(h * 64 + step, 16)] = r
                    r = r + pb[pl.ds(h * 64 + step + off, 16)]
                svec = jnp.where(lanes == h, r[0], svec)
            scv = jnp.exp(jnp.clip(svec * SCALE, -10.0, 10.0))
            zrow = jnp.where(lanes < H, scv, 0.0)
            ztmp[pl.ds(8, 16)] = zrow
            zrow_hi = ztmp[pl.ds(0, 16)]
            o = jnp.bitwise_and(d, 15) * H
            hi = o > D - 16
            off = jnp.where(hi, D - 16, o)
            hf = jnp.where(hi, 1.0, 0.0)
            zrows[e, pl.ds(off, 16)] = zrow + (zrow_hi - zrow) * hf
            for h in range(H):
                sch = scv[h]
                vvec = kvrows_b[e, pl.ds(D + h * DK, 16)]
                wvrows[e, pl.ds(h * DK, 16)] = vvec * jnp.broadcast_to(sch, (16,))
            return 0
        lax.fori_loop(0, B, _edge, 0)
        pltpu.async_copy(wvrows, acc_wv.at[sdst], wsem, add=True)
        pltpu.async_copy(zrows, acc_z.at[szidx], zsem, add=True)

    _issue_idx(0, 0)
    _wait_idx(0)
    _issue_gather(0)
    _issue_idx(1, 1)

    def _super(g, _):
        for b in range(2):
            j = 2 * g + b
            @pl.when(j + 1 < NB)
            def _():
                _wait_idx(1 - b)
                _issue_gather(1 - b)
            _wait_gather(b)
            @pl.when(j >= 1)
            def _():
                _wait_scatter()
            _derive(b)
            @pl.when(j + 2 < NB)
            def _():
                _issue_idx(b, j + 2)
            _compute_scatter(b, j)
        return 0

    lax.fori_loop(0, NB // 2, _super, 0)
    _wait_scatter()
    plsc.subcore_barrier()

    def _dump_wv(t, _):
        r0 = sid * (N_PAD // NS) + t * RC
        pltpu.sync_copy(acc_wv.at[pl.ds(r0, RC)], chunk)
        pltpu.sync_copy(chunk, wv_hbm.at[cid, pl.ds(r0, RC)])
        return 0
    lax.fori_loop(0, N_PAD // NS // RC, _dump_wv, 0)

    def _dump_z(t, _):
        r0 = sid * (NZ // NS) + t * RC
        pltpu.sync_copy(acc_z.at[pl.ds(r0, RC)], chunk)
        pltpu.sync_copy(chunk, z_hbm.at[cid, pl.ds(r0, RC)])
        return 0
    lax.fori_loop(0, NZ // NS // RC, _dump_z, 0)


@functools.lru_cache(maxsize=1)
def _edge_attention():
    return pl.kernel(
        _edge_body,
        out_type=(jax.ShapeDtypeStruct((NC, N_PAD, D), jnp.float32),
                  jax.ShapeDtypeStruct((NC, NZ, D), jnp.float32)),
        mesh=plsc.VectorSubcoreMesh(core_axis_name="c", subcore_axis_name="s",
                                    num_cores=NC, num_subcores=NS),
        scratch_types=(
            [pltpu.VMEM((B,), jnp.int32),
             pltpu.VMEM((B,), jnp.int32),
             pltpu.VMEM((B + 16,), jnp.int32),
             pltpu.VMEM((B,), jnp.int32),
             pltpu.VMEM((B, 2 * D), jnp.float32),
             pltpu.VMEM((B, D), jnp.float32)] * 2
            + [
                pltpu.VMEM((B,), jnp.int32),
                pltpu.VMEM((B,), jnp.int32),
                pltpu.VMEM((B, D), jnp.float32),
                pltpu.VMEM((B, D), jnp.float32),
                pltpu.VMEM((RC, D), jnp.float32),
                pltpu.VMEM((8 * 64 + 16,), jnp.float32),
                pltpu.VMEM((32,), jnp.float32),
                pltpu.VMEM_SHARED((N_PAD, D), jnp.float32),
                pltpu.VMEM_SHARED((NZ, D), jnp.float32),
                pltpu.SemaphoreType.DMA,
                pltpu.SemaphoreType.DMA,
                pltpu.SemaphoreType.DMA,
                pltpu.SemaphoreType.DMA,
                pltpu.SemaphoreType.DMA,
                pltpu.SemaphoreType.DMA,
                pltpu.SemaphoreType.DMA,
                pltpu.SemaphoreType.DMA,
            ]),
    )


def kernel(tgt_values, tgt_positions, edge_src, edge_dst, coord_tab, pos_tab,
           val_tab, ln1_s, ln1_b, Wq, Wk, Wv, Wo, ln2_s, ln2_b, W1, W2,
           gen_ln_s, gen_ln_b, Wgen):
    vals2 = tgt_values.astype(jnp.int32).reshape(N, 1)
    pos2 = tgt_positions.astype(jnp.int32).reshape(N, 1)
    src = edge_src.astype(jnp.int32)
    dst = edge_dst.astype(jnp.int32)
    coordp = jnp.zeros((8, D), jnp.float32).at[0:3].set(coord_tab)

    x0, q0, kv0 = _embed_qkv(
        vals2, pos2, coordp, pos_tab, val_tab,
        ln1_s[0].reshape(1, D), ln1_b[0].reshape(1, D), Wq[0], Wk[0], Wv[0])
    edge_attention = _edge_attention()
    wv0, z0 = edge_attention(q0, kv0, src, dst)
    zp0 = z0.reshape(NC, N_PAD, H)
    x1, q1, kv1 = _post_qkv(
        x0, wv0, zp0, Wo[0], ln2_s[0].reshape(1, D), ln2_b[0].reshape(1, D),
        W1[0], W2[0],
        ln1_s[1].reshape(1, D), ln1_b[1].reshape(1, D), Wq[1], Wk[1], Wv[1])
    wv1, z1 = edge_attention(q1, kv1, src, dst)
    zp1 = z1.reshape(NC, N_PAD, H)
    out = _post_gen(
        x1, wv1, zp1, Wo[1], ln2_s[1].reshape(1, D), ln2_b[1].reshape(1, D),
        W1[1], W2[1],
        gen_ln_s.reshape(1, D), gen_ln_b.reshape(1, D), Wgen)
    return out


# speedup vs baseline: 46.8160x; 2.3221x over previous
import functools

import jax
import jax.numpy as jnp
from jax import lax
from jax.experimental import pallas as pl
from jax.experimental.pallas import tpu as pltpu
from jax.experimental.pallas import tpu_sc as plsc

N = 10000
E = 320000
D = 128
H = 8
DK = 16
V = 512
P = 64
FF = 512

R = 400
GRID = N // R

NC = 2
NS = 16
NW = NC * NS
EW = E // NW
B = 40
NB = EW // B
N_PAD = 10240
NZ = N_PAD // 16
RC = 8
SCALE = 1.0 / (DK ** 0.5)


def _ln(x, s, b):
    m = jnp.mean(x, axis=-1, keepdims=True)
    var = jnp.mean(x * x, axis=-1, keepdims=True) - m * m
    return (x - m) * jax.lax.rsqrt(var + 1e-5) * s + b


def _qkv(xn, wq_ref, wk_ref, wv_ref, q_out, kv_out):
    q_out[...] = jnp.dot(xn, wq_ref[...], preferred_element_type=jnp.float32)
    kv_out[:, 0:D] = jnp.dot(xn, wk_ref[...], preferred_element_type=jnp.float32)
    kv_out[:, D:2 * D] = jnp.dot(xn, wv_ref[...], preferred_element_type=jnp.float32)


def _post(x, wv_ref, z_ref, wo_ref, ln2s_ref, ln2b_ref, w1_ref, w2_ref):
    wv = wv_ref[0] + wv_ref[1]
    z = z_ref[0] + z_ref[1]
    r = 1.0 / (z + 1e-9)
    col = lax.broadcasted_iota(jnp.int32, (H, D), 1) // DK
    row = lax.broadcasted_iota(jnp.int32, (H, D), 0)
    expand = (col == row).astype(jnp.float32)
    rrep = jnp.dot(r, expand, preferred_element_type=jnp.float32)
    o = jnp.dot(wv * rrep, wo_ref[...], preferred_element_type=jnp.float32)
    x1 = x + o
    xn2 = _ln(x1, ln2s_ref[...], ln2b_ref[...])
    f = jnp.dot(
        jax.nn.relu(jnp.dot(xn2, w1_ref[...], preferred_element_type=jnp.float32)),
        w2_ref[...], preferred_element_type=jnp.float32)
    return x1 + f


def _embed_qkv_body(vals_ref, pos_ref, coord_ref, ptab_ref, vtab_ref,
                    ln1s_ref, ln1b_ref, wq_ref, wk_ref, wv_ref,
                    x_out, q_out, kv_out):
    vals = vals_ref[...]
    pos = pos_ref[...]
    c = pos % 3
    p = pos // 3
    x = jnp.where(c == 0, coord_ref[0:1, :],
                  jnp.where(c == 1, coord_ref[1:2, :], coord_ref[2:3, :]))
    oh_p = (p == lax.broadcasted_iota(jnp.int32, (R, P), 1)).astype(jnp.float32)
    x = x + jnp.dot(oh_p, ptab_ref[...], preferred_element_type=jnp.float32)
    oh_v = (vals == lax.broadcasted_iota(jnp.int32, (R, V), 1)).astype(jnp.float32)
    x = x + jnp.dot(oh_v, vtab_ref[...], preferred_element_type=jnp.float32)
    x_out[...] = x
    xn = _ln(x, ln1s_ref[...], ln1b_ref[...])
    _qkv(xn, wq_ref, wk_ref, wv_ref, q_out, kv_out)


def _post_qkv_body(x_ref, wv_ref, z_ref, wo_ref, ln2s_ref, ln2b_ref,
                   w1_ref, w2_ref, ln1s_ref, ln1b_ref, wq_ref, wk_ref, wv2_ref,
                   x_out, q_out, kv_out):
    x2 = _post(x_ref[...], wv_ref, z_ref, wo_ref, ln2s_ref, ln2b_ref,
               w1_ref, w2_ref)
    x_out[...] = x2
    xn = _ln(x2, ln1s_ref[...], ln1b_ref[...])
    _qkv(xn, wq_ref, wk_ref, wv2_ref, q_out, kv_out)


def _post_gen_body(x_ref, wv_ref, z_ref, wo_ref, ln2s_ref, ln2b_ref,
                   w1_ref, w2_ref, glns_ref, glnb_ref, wgen_ref, out_ref):
    x2 = _post(x_ref[...], wv_ref, z_ref, wo_ref, ln2s_ref, ln2b_ref,
               w1_ref, w2_ref)
    xg = _ln(x2, glns_ref[...], glnb_ref[...])
    logits = jnp.dot(xg, wgen_ref[...], preferred_element_type=jnp.float32)
    m = jnp.max(logits, axis=-1, keepdims=True)
    lse = m + jnp.log(jnp.sum(jnp.exp(logits - m), axis=-1, keepdims=True))
    out_ref[...] = logits - lse


def _full(shape):
    return pl.BlockSpec(shape, lambda i: tuple(0 for _ in shape))


_ROW = pl.BlockSpec((R, D), lambda i: (i, 0))
_ROW_KV = pl.BlockSpec((R, 2 * D), lambda i: (i, 0))
_ROW_WV = pl.BlockSpec((NC, R, D), lambda i: (0, i, 0))
_ROW_Z = pl.BlockSpec((NC, R, H), lambda i: (0, i, 0))
_ROW_IDX = pl.BlockSpec((R, 1), lambda i: (i, 0))

_embed_qkv = pl.pallas_call(
    _embed_qkv_body,
    grid=(GRID,),
    in_specs=[_ROW_IDX, _ROW_IDX, _full((8, D)), _full((P, D)), _full((V, D)),
              _full((1, D)), _full((1, D)),
              _full((D, D)), _full((D, D)), _full((D, D))],
    out_specs=[_ROW, _ROW, _ROW_KV],
    out_shape=[jax.ShapeDtypeStruct((N, D), jnp.float32),
               jax.ShapeDtypeStruct((N, D), jnp.float32),
               jax.ShapeDtypeStruct((N, 2 * D), jnp.float32)],
)

_post_qkv = pl.pallas_call(
    _post_qkv_body,
    grid=(GRID,),
    in_specs=[_ROW, _ROW_WV, _ROW_Z, _full((D, D)), _full((1, D)), _full((1, D)),
              _full((D, FF)), _full((FF, D)),
              _full((1, D)), _full((1, D)),
              _full((D, D)), _full((D, D)), _full((D, D))],
    out_specs=[_ROW, _ROW, _ROW_KV],
    out_shape=[jax.ShapeDtypeStruct((N, D), jnp.float32),
               jax.ShapeDtypeStruct((N, D), jnp.float32),
               jax.ShapeDtypeStruct((N, 2 * D), jnp.float32)],
)

_post_gen = pl.pallas_call(
    _post_gen_body,
    grid=(GRID,),
    in_specs=[_ROW, _ROW_WV, _ROW_Z, _full((D, D)), _full((1, D)), _full((1, D)),
              _full((D, FF)), _full((FF, D)),
              _full((1, D)), _full((1, D)), _full((D, V))],
    out_specs=pl.BlockSpec((R, V), lambda i: (i, 0)),
    out_shape=jax.ShapeDtypeStruct((N, V), jnp.float32),
)


def _edge_body(q_hbm, kv_hbm, src_hbm, dst_hbm, wv_hbm, z_hbm,
               src_idx0, dst_idx0, dst_pad0, zidx0, kvrows0, qrows0,
               src_idx1, dst_idx1, dst_pad1, zidx1, kvrows1, qrows1,
               sdst, szidx, wvrows, zrows, chunk, ztmp,
               acc_wv, acc_z,
               ksem0, qsem0, ksem1, qsem1, isem0, isem1, wsem, zsem):
    cid = lax.axis_index("c")
    sid = lax.axis_index("s")
    wid = cid * NS + sid
    zeros16 = jnp.zeros((16,), jnp.float32)
    lanes = lax.broadcasted_iota(jnp.int32, (16,), 0)

    bufs = ((src_idx0, dst_idx0, dst_pad0, zidx0, kvrows0, qrows0,
             ksem0, qsem0, isem0),
            (src_idx1, dst_idx1, dst_pad1, zidx1, kvrows1, qrows1,
             ksem1, qsem1, isem1))

    def _zero_row(rr, _):
        for cc in range(D // 16):
            chunk[rr, pl.ds(cc * 16, 16)] = zeros16
        return 0
    lax.fori_loop(0, RC, _zero_row, 0)

    def _zero_wv(t, _):
        pltpu.sync_copy(chunk, acc_wv.at[pl.ds(sid * (N_PAD // NS) + t * RC, RC)])
        return 0
    lax.fori_loop(0, N_PAD // NS // RC, _zero_wv, 0)

    def _zero_z(t, _):
        pltpu.sync_copy(chunk, acc_z.at[pl.ds(sid * (NZ // NS) + t * RC, RC)])
        return 0
    lax.fori_loop(0, NZ // NS // RC, _zero_z, 0)
    ztmp[pl.ds(0, 16)] = zeros16
    ztmp[pl.ds(16, 16)] = zeros16
    plsc.subcore_barrier()

    def _issue_idx(b, j):
        base = wid * EW + j * B
        pltpu.async_copy(src_hbm.at[pl.ds(base, B)], bufs[b][0], bufs[b][8])
        pltpu.async_copy(dst_hbm.at[pl.ds(base, B)], bufs[b][1], bufs[b][8])

    def _wait_idx(b):
        pltpu.make_async_copy(src_hbm.at[pl.ds(0, B)], bufs[b][0], bufs[b][8]).wait()
        pltpu.make_async_copy(dst_hbm.at[pl.ds(0, B)], bufs[b][1], bufs[b][8]).wait()

    def _issue_gather(b):
        pltpu.async_copy(kv_hbm.at[bufs[b][0]], bufs[b][4], bufs[b][6])
        pltpu.async_copy(q_hbm.at[bufs[b][1]], bufs[b][5], bufs[b][7])

    def _wait_gather(b):
        pltpu.make_async_copy(kv_hbm.at[bufs[b][0]], bufs[b][4], bufs[b][6]).wait()
        pltpu.make_async_copy(q_hbm.at[bufs[b][1]], bufs[b][5], bufs[b][7]).wait()

    def _wait_scatter():
        pltpu.make_async_copy(wvrows, acc_wv.at[sdst], wsem).wait()
        pltpu.make_async_copy(zrows, acc_z.at[szidx], zsem).wait()

    def _derive(b):
        dst_b, pad_b = bufs[b][1], bufs[b][2]
        for c in range(-(-B // 16)):
            cc = min(c * 16, B - 16)
            dv = dst_b[pl.ds(cc, 16)]
            sdst[pl.ds(cc, 16)] = dv
            szidx[pl.ds(cc, 16)] = lax.shift_right_logical(dv, 4)
            pad_b[pl.ds(cc, 16)] = dv
        pad_b[pl.ds(B, 16)] = jnp.zeros((16,), jnp.int32)

    def _compute_scatter(b, j):
        pad_b, kvrows_b, qrows_b = bufs[b][2], bufs[b][4], bufs[b][5]

        def _zrow_zero(e, _):
            for cc in range(D // 16):
                zrows[e, pl.ds(cc * 16, 16)] = zeros16
            return 0
        lax.fori_loop(0, B, _zrow_zero, 0)

        def _edge(e, _):
            d = pad_b[pl.ds(e, 16)][0]
            svec = zeros16
            for h in range(H):
                kvec = kvrows_b[e, pl.ds(h * DK, 16)]
                qvec = qrows_b[e, pl.ds(h * DK, 16)]
                r = kvec * qvec
                for k in (8, 4, 2, 1):
                    r = r + r[jnp.bitwise_xor(lanes, k)]
                svec = jnp.where(lanes == h, r, svec)
            scv = jnp.exp(jnp.clip(svec * SCALE, -10.0, 10.0))
            zrow = jnp.where(lanes < H, scv, 0.0)
            ztmp[pl.ds(8, 16)] = zrow
            zrow_hi = ztmp[pl.ds(0, 16)]
            o = jnp.bitwise_and(d, 15) * H
            hi = o > D - 16
            off = jnp.where(hi, D - 16, o)
            hf = jnp.where(hi, 1.0, 0.0)
            zrows[e, pl.ds(off, 16)] = zrow + (zrow_hi - zrow) * hf
            for h in range(H):
                sch = scv[h]
                vvec = kvrows_b[e, pl.ds(D + h * DK, 16)]
                wvrows[e, pl.ds(h * DK, 16)] = vvec * jnp.broadcast_to(sch, (16,))
            return 0
        lax.fori_loop(0, B, _edge, 0)
        pltpu.async_copy(wvrows, acc_wv.at[sdst], wsem, add=True)
        pltpu.async_copy(zrows, acc_z.at[szidx], zsem, add=True)

    _issue_idx(0, 0)
    _wait_idx(0)
    _issue_gather(0)
    _issue_idx(1, 1)

    def _super(g, _):
        for b in range(2):
            j = 2 * g + b
            @pl.when(j + 1 < NB)
            def _():
                _wait_idx(1 - b)
                _issue_gather(1 - b)
            _wait_gather(b)
            @pl.when(j >= 1)
            def _():
                _wait_scatter()
            _derive(b)
            @pl.when(j + 2 < NB)
            def _():
                _issue_idx(b, j + 2)
            _compute_scatter(b, j)
        return 0

    lax.fori_loop(0, NB // 2, _super, 0)
    _wait_scatter()
    plsc.subcore_barrier()

    def _dump_wv(t, _):
        r0 = sid * (N_PAD // NS) + t * RC
        pltpu.sync_copy(acc_wv.at[pl.ds(r0, RC)], chunk)
        pltpu.sync_copy(chunk, wv_hbm.at[cid, pl.ds(r0, RC)])
        return 0
    lax.fori_loop(0, N_PAD // NS // RC, _dump_wv, 0)

    def _dump_z(t, _):
        r0 = sid * (NZ // NS) + t * RC
        pltpu.sync_copy(acc_z.at[pl.ds(r0, RC)], chunk)
        pltpu.sync_copy(chunk, z_hbm.at[cid, pl.ds(r0, RC)])
        return 0
    lax.fori_loop(0, NZ // NS // RC, _dump_z, 0)


@functools.lru_cache(maxsize=1)
def _edge_attention():
    return pl.kernel(
        _edge_body,
        out_type=(jax.ShapeDtypeStruct((NC, N_PAD, D), jnp.float32),
                  jax.ShapeDtypeStruct((NC, NZ, D), jnp.float32)),
        mesh=plsc.VectorSubcoreMesh(core_axis_name="c", subcore_axis_name="s",
                                    num_cores=NC, num_subcores=NS),
        scratch_types=(
            [pltpu.VMEM((B,), jnp.int32),
             pltpu.VMEM((B,), jnp.int32),
             pltpu.VMEM((B + 16,), jnp.int32),
             pltpu.VMEM((B,), jnp.int32),
             pltpu.VMEM((B, 2 * D), jnp.float32),
             pltpu.VMEM((B, D), jnp.float32)] * 2
            + [
                pltpu.VMEM((B,), jnp.int32),
                pltpu.VMEM((B,), jnp.int32),
                pltpu.VMEM((B, D), jnp.float32),
                pltpu.VMEM((B, D), jnp.float32),
                pltpu.VMEM((RC, D), jnp.float32),
                pltpu.VMEM((32,), jnp.float32),
                pltpu.VMEM_SHARED((N_PAD, D), jnp.float32),
                pltpu.VMEM_SHARED((NZ, D), jnp.float32),
                pltpu.SemaphoreType.DMA,
                pltpu.SemaphoreType.DMA,
                pltpu.SemaphoreType.DMA,
                pltpu.SemaphoreType.DMA,
                pltpu.SemaphoreType.DMA,
                pltpu.SemaphoreType.DMA,
                pltpu.SemaphoreType.DMA,
                pltpu.SemaphoreType.DMA,
            ]),
    )


def kernel(tgt_values, tgt_positions, edge_src, edge_dst, coord_tab, pos_tab,
           val_tab, ln1_s, ln1_b, Wq, Wk, Wv, Wo, ln2_s, ln2_b, W1, W2,
           gen_ln_s, gen_ln_b, Wgen):
    vals2 = tgt_values.astype(jnp.int32).reshape(N, 1)
    pos2 = tgt_positions.astype(jnp.int32).reshape(N, 1)
    src = edge_src.astype(jnp.int32)
    dst = edge_dst.astype(jnp.int32)
    coordp = jnp.zeros((8, D), jnp.float32).at[0:3].set(coord_tab)

    x0, q0, kv0 = _embed_qkv(
        vals2, pos2, coordp, pos_tab, val_tab,
        ln1_s[0].reshape(1, D), ln1_b[0].reshape(1, D), Wq[0], Wk[0], Wv[0])
    edge_attention = _edge_attention()
    wv0, z0 = edge_attention(q0, kv0, src, dst)
    zp0 = z0.reshape(NC, N_PAD, H)
    x1, q1, kv1 = _post_qkv(
        x0, wv0, zp0, Wo[0], ln2_s[0].reshape(1, D), ln2_b[0].reshape(1, D),
        W1[0], W2[0],
        ln1_s[1].reshape(1, D), ln1_b[1].reshape(1, D), Wq[1], Wk[1], Wv[1])
    wv1, z1 = edge_attention(q1, kv1, src, dst)
    zp1 = z1.reshape(NC, N_PAD, H)
    out = _post_gen(
        x1, wv1, zp1, Wo[1], ln2_s[1].reshape(1, D), ln2_b[1].reshape(1, D),
        W1[1], W2[1],
        gen_ln_s.reshape(1, D), gen_ln_b.reshape(1, D), Wgen)
    return out


# speedup vs baseline: 47.1216x; 1.0065x over previous
import functools

import jax
import jax.numpy as jnp
from jax import lax
from jax.experimental import pallas as pl
from jax.experimental.pallas import tpu as pltpu
from jax.experimental.pallas import tpu_sc as plsc

N = 10000
E = 320000
D = 128
H = 8
DK = 16
V = 512
P = 64
FF = 512

R = 400
GRID = N // R

NC = 2
NS = 16
NW = NC * NS
EW = E // NW
B = 40
NB = EW // B
N_PAD = 10240
NZ = N_PAD // 16
RC = 8
SCALE = 1.0 / (DK ** 0.5)


def _ln(x, s, b):
    m = jnp.mean(x, axis=-1, keepdims=True)
    var = jnp.mean(x * x, axis=-1, keepdims=True) - m * m
    return (x - m) * jax.lax.rsqrt(var + 1e-5) * s + b


def _qkv(xn, wq_ref, wk_ref, wv_ref, q_out, kv_out):
    q_out[...] = jnp.dot(xn, wq_ref[...], preferred_element_type=jnp.float32)
    kv_out[:, 0:D] = jnp.dot(xn, wk_ref[...], preferred_element_type=jnp.float32)
    kv_out[:, D:2 * D] = jnp.dot(xn, wv_ref[...], preferred_element_type=jnp.float32)


def _post(x, wv_ref, z_ref, wo_ref, ln2s_ref, ln2b_ref, w1_ref, w2_ref):
    wv = wv_ref[0] + wv_ref[1]
    z = z_ref[0] + z_ref[1]
    r = 1.0 / (z + 1e-9)
    col = lax.broadcasted_iota(jnp.int32, (H, D), 1) // DK
    row = lax.broadcasted_iota(jnp.int32, (H, D), 0)
    expand = (col == row).astype(jnp.float32)
    rrep = jnp.dot(r, expand, preferred_element_type=jnp.float32)
    o = jnp.dot(wv * rrep, wo_ref[...], preferred_element_type=jnp.float32)
    x1 = x + o
    xn2 = _ln(x1, ln2s_ref[...], ln2b_ref[...])
    f = jnp.dot(
        jax.nn.relu(jnp.dot(xn2, w1_ref[...], preferred_element_type=jnp.float32)),
        w2_ref[...], preferred_element_type=jnp.float32)
    return x1 + f


def _embed_qkv_body(vals_ref, pos_ref, coord_ref, ptab_ref, vtab_ref,
                    ln1s_ref, ln1b_ref, wq_ref, wk_ref, wv_ref,
                    x_out, q_out, kv_out):
    vals = vals_ref[...]
    pos = pos_ref[...]
    c = pos % 3
    p = pos // 3
    x = jnp.where(c == 0, coord_ref[0:1, :],
                  jnp.where(c == 1, coord_ref[1:2, :], coord_ref[2:3, :]))
    oh_p = (p == lax.broadcasted_iota(jnp.int32, (R, P), 1)).astype(jnp.float32)
    x = x + jnp.dot(oh_p, ptab_ref[...], preferred_element_type=jnp.float32)
    oh_v = (vals == lax.broadcasted_iota(jnp.int32, (R, V), 1)).astype(jnp.float32)
    x = x + jnp.dot(oh_v, vtab_ref[...], preferred_element_type=jnp.float32)
    x_out[...] = x
    xn = _ln(x, ln1s_ref[...], ln1b_ref[...])
    _qkv(xn, wq_ref, wk_ref, wv_ref, q_out, kv_out)


def _post_qkv_body(x_ref, wv_ref, z_ref, wo_ref, ln2s_ref, ln2b_ref,
                   w1_ref, w2_ref, ln1s_ref, ln1b_ref, wq_ref, wk_ref, wv2_ref,
                   x_out, q_out, kv_out):
    x2 = _post(x_ref[...], wv_ref, z_ref, wo_ref, ln2s_ref, ln2b_ref,
               w1_ref, w2_ref)
    x_out[...] = x2
    xn = _ln(x2, ln1s_ref[...], ln1b_ref[...])
    _qkv(xn, wq_ref, wk_ref, wv2_ref, q_out, kv_out)


def _post_gen_body(x_ref, wv_ref, z_ref, wo_ref, ln2s_ref, ln2b_ref,
                   w1_ref, w2_ref, glns_ref, glnb_ref, wgen_ref, out_ref):
    x2 = _post(x_ref[...], wv_ref, z_ref, wo_ref, ln2s_ref, ln2b_ref,
               w1_ref, w2_ref)
    xg = _ln(x2, glns_ref[...], glnb_ref[...])
    logits = jnp.dot(xg, wgen_ref[...], preferred_element_type=jnp.float32)
    m = jnp.max(logits, axis=-1, keepdims=True)
    lse = m + jnp.log(jnp.sum(jnp.exp(logits - m), axis=-1, keepdims=True))
    out_ref[...] = logits - lse


def _full(shape):
    return pl.BlockSpec(shape, lambda i: tuple(0 for _ in shape))


_ROW = pl.BlockSpec((R, D), lambda i: (i, 0))
_ROW_KV = pl.BlockSpec((R, 2 * D), lambda i: (i, 0))
_ROW_WV = pl.BlockSpec((NC, R, D), lambda i: (0, i, 0))
_ROW_Z = pl.BlockSpec((NC, R, H), lambda i: (0, i, 0))
_ROW_IDX = pl.BlockSpec((R, 1), lambda i: (i, 0))

_embed_qkv = pl.pallas_call(
    _embed_qkv_body,
    grid=(GRID,),
    in_specs=[_ROW_IDX, _ROW_IDX, _full((8, D)), _full((P, D)), _full((V, D)),
              _full((1, D)), _full((1, D)),
              _full((D, D)), _full((D, D)), _full((D, D))],
    out_specs=[_ROW, _ROW, _ROW_KV],
    out_shape=[jax.ShapeDtypeStruct((N, D), jnp.float32),
               jax.ShapeDtypeStruct((N, D), jnp.float32),
               jax.ShapeDtypeStruct((N, 2 * D), jnp.float32)],
)

_post_qkv = pl.pallas_call(
    _post_qkv_body,
    grid=(GRID,),
    in_specs=[_ROW, _ROW_WV, _ROW_Z, _full((D, D)), _full((1, D)), _full((1, D)),
              _full((D, FF)), _full((FF, D)),
              _full((1, D)), _full((1, D)),
              _full((D, D)), _full((D, D)), _full((D, D))],
    out_specs=[_ROW, _ROW, _ROW_KV],
    out_shape=[jax.ShapeDtypeStruct((N, D), jnp.float32),
               jax.ShapeDtypeStruct((N, D), jnp.float32),
               jax.ShapeDtypeStruct((N, 2 * D), jnp.float32)],
)

_post_gen = pl.pallas_call(
    _post_gen_body,
    grid=(GRID,),
    in_specs=[_ROW, _ROW_WV, _ROW_Z, _full((D, D)), _full((1, D)), _full((1, D)),
              _full((D, FF)), _full((FF, D)),
              _full((1, D)), _full((1, D)), _full((D, V))],
    out_specs=pl.BlockSpec((R, V), lambda i: (i, 0)),
    out_shape=jax.ShapeDtypeStruct((N, V), jnp.float32),
)


def _edge_body(q_hbm, kv_hbm, src_hbm, dst_hbm, wv_hbm, z_hbm,
               src_idx0, dst_idx0, dst_pad0, zidx0, kvrows0, qrows0,
               src_idx1, dst_idx1, dst_pad1, zidx1, kvrows1, qrows1,
               sidx, wvzrows, chunk, ztmp,
               acc,
               ksem0, qsem0, ksem1, qsem1, isem0, isem1, wsem):
    cid = lax.axis_index("c")
    sid = lax.axis_index("s")
    wid = cid * NS + sid
    zeros16 = jnp.zeros((16,), jnp.float32)
    lanes = lax.broadcasted_iota(jnp.int32, (16,), 0)

    bufs = ((src_idx0, dst_idx0, dst_pad0, zidx0, kvrows0, qrows0,
             ksem0, qsem0, isem0),
            (src_idx1, dst_idx1, dst_pad1, zidx1, kvrows1, qrows1,
             ksem1, qsem1, isem1))

    def _zero_row(rr, _):
        for cc in range(D // 16):
            chunk[rr, pl.ds(cc * 16, 16)] = zeros16
        return 0
    lax.fori_loop(0, RC, _zero_row, 0)

    def _zero_acc(t, _):
        pltpu.sync_copy(chunk, acc.at[pl.ds(sid * ((N_PAD + NZ) // NS) + t * RC, RC)])
        return 0
    lax.fori_loop(0, (N_PAD + NZ) // NS // RC, _zero_acc, 0)
    ztmp[pl.ds(0, 16)] = zeros16
    ztmp[pl.ds(16, 16)] = zeros16
    plsc.subcore_barrier()

    def _issue_idx(b, j):
        base = wid * EW + j * B
        pltpu.async_copy(src_hbm.at[pl.ds(base, B)], bufs[b][0], bufs[b][8])
        pltpu.async_copy(dst_hbm.at[pl.ds(base, B)], bufs[b][1], bufs[b][8])

    def _wait_idx(b):
        pltpu.make_async_copy(src_hbm.at[pl.ds(0, B)], bufs[b][0], bufs[b][8]).wait()
        pltpu.make_async_copy(dst_hbm.at[pl.ds(0, B)], bufs[b][1], bufs[b][8]).wait()

    def _issue_gather(b):
        pltpu.async_copy(kv_hbm.at[bufs[b][0]], bufs[b][4], bufs[b][6])
        pltpu.async_copy(q_hbm.at[bufs[b][1]], bufs[b][5], bufs[b][7])

    def _wait_gather(b):
        pltpu.make_async_copy(kv_hbm.at[bufs[b][0]], bufs[b][4], bufs[b][6]).wait()
        pltpu.make_async_copy(q_hbm.at[bufs[b][1]], bufs[b][5], bufs[b][7]).wait()

    def _wait_scatter():
        pltpu.make_async_copy(wvzrows, acc.at[sidx], wsem).wait()

    def _derive(b):
        dst_b, pad_b = bufs[b][1], bufs[b][2]
        for c in range(-(-B // 16)):
            cc = min(c * 16, B - 16)
            dv = dst_b[pl.ds(cc, 16)]
            sidx[pl.ds(cc, 16)] = dv
            sidx[pl.ds(B + cc, 16)] = N_PAD + lax.shift_right_logical(dv, 4)
            pad_b[pl.ds(cc, 16)] = dv
        pad_b[pl.ds(B, 16)] = jnp.zeros((16,), jnp.int32)

    def _compute_scatter(b, j):
        pad_b, kvrows_b, qrows_b = bufs[b][2], bufs[b][4], bufs[b][5]

        def _zrow_zero(e, _):
            for cc in range(D // 16):
                wvzrows[e, pl.ds(cc * 16, 16)] = zeros16
            return 0
        lax.fori_loop(B, 2 * B, _zrow_zero, 0)

        def _edge(e, _):
            d = pad_b[pl.ds(e, 16)][0]
            svec = zeros16
            for h in range(H):
                kvec = kvrows_b[e, pl.ds(h * DK, 16)]
                qvec = qrows_b[e, pl.ds(h * DK, 16)]
                r = kvec * qvec
                for k in (8, 4, 2, 1):
                    r = r + r[jnp.bitwise_xor(lanes, k)]
                svec = jnp.where(lanes == h, r, svec)
            scv = jnp.exp(jnp.clip(svec * SCALE, -10.0, 10.0))
            zrow = jnp.where(lanes < H, scv, 0.0)
            ztmp[pl.ds(8, 16)] = zrow
            zrow_hi = ztmp[pl.ds(0, 16)]
            o = jnp.bitwise_and(d, 15) * H
            hi = o > D - 16
            off = jnp.where(hi, D - 16, o)
            hf = jnp.where(hi, 1.0, 0.0)
            wvzrows[B + e, pl.ds(off, 16)] = zrow + (zrow_hi - zrow) * hf
            for h in range(H):
                sch = scv[h]
                vvec = kvrows_b[e, pl.ds(D + h * DK, 16)]
                wvzrows[e, pl.ds(h * DK, 16)] = vvec * jnp.broadcast_to(sch, (16,))
            return 0
        lax.fori_loop(0, B, _edge, 0)
        pltpu.async_copy(wvzrows, acc.at[sidx], wsem, add=True)

    _issue_idx(0, 0)
    _wait_idx(0)
    _issue_gather(0)
    _issue_idx(1, 1)

    def _super(g, _):
        for b in range(2):
            j = 2 * g + b
            @pl.when(j + 1 < NB)
            def _():
                _wait_idx(1 - b)
                _issue_gather(1 - b)
            _wait_gather(b)
            @pl.when(j >= 1)
            def _():
                _wait_scatter()
            _derive(b)
            @pl.when(j + 2 < NB)
            def _():
                _issue_idx(b, j + 2)
            _compute_scatter(b, j)
        return 0

    lax.fori_loop(0, NB // 2, _super, 0)
    _wait_scatter()
    plsc.subcore_barrier()

    def _dump_wv(t, _):
        r0 = sid * (N_PAD // NS) + t * RC
        pltpu.sync_copy(acc.at[pl.ds(r0, RC)], chunk)
        pltpu.sync_copy(chunk, wv_hbm.at[cid, pl.ds(r0, RC)])
        return 0
    lax.fori_loop(0, N_PAD // NS // RC, _dump_wv, 0)

    def _dump_z(t, _):
        r0 = sid * (NZ // NS) + t * RC
        pltpu.sync_copy(acc.at[pl.ds(N_PAD + r0, RC)], chunk)
        pltpu.sync_copy(chunk, z_hbm.at[cid, pl.ds(r0, RC)])
        return 0
    lax.fori_loop(0, NZ // NS // RC, _dump_z, 0)


@functools.lru_cache(maxsize=1)
def _edge_attention():
    return pl.kernel(
        _edge_body,
        out_type=(jax.ShapeDtypeStruct((NC, N_PAD, D), jnp.float32),
                  jax.ShapeDtypeStruct((NC, NZ, D), jnp.float32)),
        mesh=plsc.VectorSubcoreMesh(core_axis_name="c", subcore_axis_name="s",
                                    num_cores=NC, num_subcores=NS),
        scratch_types=(
            [pltpu.VMEM((B,), jnp.int32),
             pltpu.VMEM((B,), jnp.int32),
             pltpu.VMEM((B + 16,), jnp.int32),
             pltpu.VMEM((B,), jnp.int32),
             pltpu.VMEM((B, 2 * D), jnp.float32),
             pltpu.VMEM((B, D), jnp.float32)] * 2
            + [
                pltpu.VMEM((2 * B,), jnp.int32),
                pltpu.VMEM((2 * B, D), jnp.float32),
                pltpu.VMEM((RC, D), jnp.float32),
                pltpu.VMEM((32,), jnp.float32),
                pltpu.VMEM_SHARED((N_PAD + NZ, D), jnp.float32),
                pltpu.SemaphoreType.DMA,
                pltpu.SemaphoreType.DMA,
                pltpu.SemaphoreType.DMA,
                pltpu.SemaphoreType.DMA,
                pltpu.SemaphoreType.DMA,
                pltpu.SemaphoreType.DMA,
                pltpu.SemaphoreType.DMA,
            ]),
    )


def kernel(tgt_values, tgt_positions, edge_src, edge_dst, coord_tab, pos_tab,
           val_tab, ln1_s, ln1_b, Wq, Wk, Wv, Wo, ln2_s, ln2_b, W1, W2,
           gen_ln_s, gen_ln_b, Wgen):
    vals2 = tgt_values.astype(jnp.int32).reshape(N, 1)
    pos2 = tgt_positions.astype(jnp.int32).reshape(N, 1)
    src = edge_src.astype(jnp.int32)
    dst = edge_dst.astype(jnp.int32)
    coordp = jnp.zeros((8, D), jnp.float32).at[0:3].set(coord_tab)

    x0, q0, kv0 = _embed_qkv(
        vals2, pos2, coordp, pos_tab, val_tab,
        ln1_s[0].reshape(1, D), ln1_b[0].reshape(1, D), Wq[0], Wk[0], Wv[0])
    edge_attention = _edge_attention()
    wv0, z0 = edge_attention(q0, kv0, src, dst)
    zp0 = z0.reshape(NC, N_PAD, H)
    x1, q1, kv1 = _post_qkv(
        x0, wv0, zp0, Wo[0], ln2_s[0].reshape(1, D), ln2_b[0].reshape(1, D),
        W1[0], W2[0],
        ln1_s[1].reshape(1, D), ln1_b[1].reshape(1, D), Wq[1], Wk[1], Wv[1])
    wv1, z1 = edge_attention(q1, kv1, src, dst)
    zp1 = z1.reshape(NC, N_PAD, H)
    out = _post_gen(
        x1, wv1, zp1, Wo[1], ln2_s[1].reshape(1, D), ln2_b[1].reshape(1, D),
        W1[1], W2[1],
        gen_ln_s.reshape(1, D), gen_ln_b.reshape(1, D), Wgen)
    return out
